# Initial kernel scaffold; baseline (speedup 1.0000x reference)
import functools

import jax
import jax.numpy as jnp
from jax import lax
from jax.experimental import pallas as pl
from jax.experimental.pallas import tpu as pltpu
from jax.experimental.pallas import tpu_sc as plsc

_N = 10000
_NACC = 10240
_D = 128
_NCLS = 40
_DC = 48
_E = 320000
_EPAD = 327680
_NC = 2
_NS = 16
_NW = _NC * _NS
_EPT = _EPAD // _NW
_C = 128
_NCH = _EPT // _C
_RPT = _NACC // _NS
_L2 = 2 * _NACC
_EPT2 = 2 * _EPT
_NCH2 = _EPT2 // _C
_RPT2 = _L2 // _NS
_BR = 512
_GRID = _NACC // _BR


def _sc_mesh():
    return plsc.VectorSubcoreMesh(core_axis_name="c", subcore_axis_name="s")


@functools.lru_cache(maxsize=None)
def _make_agg(D):
    @functools.partial(
        pl.kernel,
        out_type=jax.ShapeDtypeStruct((_NC * _NACC, D), jnp.float32),
        mesh=_sc_mesh(),
        scratch_types=[
            pltpu.VMEM((_C,), jnp.int32),
            pltpu.VMEM((_C,), jnp.int32),
            pltpu.VMEM((_C, D), jnp.float32),
            pltpu.VMEM_SHARED((_NACC, D), jnp.float32),
            pltpu.SemaphoreType.DMA,
        ],
    )
    def agg(table, srcp, dstp, zeros, out, sidx, didx, rows, acc, sem):
        c = lax.axis_index("c")
        s = lax.axis_index("s")
        wid = c * _NS + s
        rbase = s * _RPT
        pltpu.sync_copy(zeros.at[pl.ds(rbase, _RPT)], acc.at[pl.ds(rbase, _RPT)])
        plsc.subcore_barrier()
        ebase = wid * _EPT

        def body(j, carry):
            off = ebase + j * _C
            pltpu.sync_copy(srcp.at[pl.ds(off, _C)], sidx)
            pltpu.sync_copy(dstp.at[pl.ds(off, _C)], didx)
            pltpu.async_copy(table.at[sidx], rows, sem).wait()
            pltpu.sync_copy(rows, acc.at[didx], add=True)
            return carry

        lax.fori_loop(0, _NCH, body, 0)
        plsc.subcore_barrier()
        pltpu.sync_copy(acc.at[pl.ds(rbase, _RPT)],
                        out.at[pl.ds(c * _NACC + rbase, _RPT)])

    return agg


@functools.partial(
    pl.kernel,
    out_type=jax.ShapeDtypeStruct((_NC * _L2, 16), jnp.float32),
    mesh=_sc_mesh(),
    scratch_types=[
        pltpu.VMEM((_C,), jnp.int32),
        pltpu.VMEM((_C, 16), jnp.float32),
        pltpu.VMEM_SHARED((_L2, 16), jnp.float32),
    ],
)
def _deg_kernel(idxs, ones, zeros, out, idxbuf, ones_v, acc):
    c = lax.axis_index("c")
    s = lax.axis_index("s")
    wid = c * _NS + s
    rbase = s * _RPT2
    pltpu.sync_copy(ones, ones_v)
    pltpu.sync_copy(zeros.at[pl.ds(rbase, _RPT2)], acc.at[pl.ds(rbase, _RPT2)])
    plsc.subcore_barrier()
    base = wid * _EPT2

    def body(j, carry):
        pltpu.sync_copy(idxs.at[pl.ds(base + j * _C, _C)], idxbuf)
        pltpu.sync_copy(ones_v, acc.at[idxbuf], add=True)
        return carry

    lax.fori_loop(0, _NCH2, body, 0)
    plsc.subcore_barrier()
    pltpu.sync_copy(acc.at[pl.ds(rbase, _RPT2)],
                    out.at[pl.ds(c * _L2 + rbase, _RPT2)])


def _tc_norm_scale(xpad, degparts):
    def body(x_ref, dp_ref, xs_ref, ns_ref, nd_ref):
        d = dp_ref[0] + dp_ref[1]
        ns = lax.rsqrt(jnp.maximum(d[0, :, 0:1], 1.0))
        nd = lax.rsqrt(jnp.maximum(d[1, :, 0:1], 1.0))
        xs_ref[...] = x_ref[...] * ns
        ns_ref[...] = ns
        nd_ref[...] = nd

    return pl.pallas_call(
        body,
        grid=(_GRID,),
        in_specs=[
            pl.BlockSpec((_BR, _D), lambda i: (i, 0)),
            pl.BlockSpec((_NC, 2, _BR, 16), lambda i: (0, 0, i, 0)),
        ],
        out_specs=[
            pl.BlockSpec((_BR, _D), lambda i: (i, 0)),
            pl.BlockSpec((_BR, 1), lambda i: (i, 0)),
            pl.BlockSpec((_BR, 1), lambda i: (i, 0)),
        ],
        out_shape=[
            jax.ShapeDtypeStruct((_NACC, _D), jnp.float32),
            jax.ShapeDtypeStruct((_NACC, 1), jnp.float32),
            jax.ShapeDtypeStruct((_NACC, 1), jnp.float32),
        ],
    )(xpad, degparts)


def _tc_layer(parts, nd, ns, W, b):
    def body(p_ref, nd_ref, ns_ref, w_ref, b_ref, o_ref):
        h = (p_ref[0] + p_ref[1]) * nd_ref[...]
        z = jnp.dot(h, w_ref[...], preferred_element_type=jnp.float32) + b_ref[...]
        o_ref[...] = jnp.where(z > 0.0, z, jnp.exp(z) - 1.0) * ns_ref[...]

    return pl.pallas_call(
        body,
        grid=(_GRID,),
        in_specs=[
            pl.BlockSpec((_NC, _BR, _D), lambda i: (0, i, 0)),
            pl.BlockSpec((_BR, 1), lambda i: (i, 0)),
            pl.BlockSpec((_BR, 1), lambda i: (i, 0)),
            pl.BlockSpec((_D, _D), lambda i: (0, 0)),
            pl.BlockSpec((1, _D), lambda i: (0, 0)),
        ],
        out_specs=pl.BlockSpec((_BR, _D), lambda i: (i, 0)),
        out_shape=jax.ShapeDtypeStruct((_NACC, _D), jnp.float32),
    )(parts, nd, ns, W, b)


def _tc_layer_premat(parts, nd, ns, W, b, W2p):
    def body(p_ref, nd_ref, ns_ref, w_ref, b_ref, w2_ref, o_ref):
        h = (p_ref[0] + p_ref[1]) * nd_ref[...]
        z = jnp.dot(h, w_ref[...], preferred_element_type=jnp.float32) + b_ref[...]
        h2 = jnp.where(z > 0.0, z, jnp.exp(z) - 1.0) * ns_ref[...]
        o_ref[...] = jnp.dot(h2, w2_ref[...], preferred_element_type=jnp.float32)

    return pl.pallas_call(
        body,
        grid=(_GRID,),
        in_specs=[
            pl.BlockSpec((_NC, _BR, _D), lambda i: (0, i, 0)),
            pl.BlockSpec((_BR, 1), lambda i: (i, 0)),
            pl.BlockSpec((_BR, 1), lambda i: (i, 0)),
            pl.BlockSpec((_D, _D), lambda i: (0, 0)),
            pl.BlockSpec((1, _D), lambda i: (0, 0)),
            pl.BlockSpec((_D, _DC), lambda i: (0, 0)),
        ],
        out_specs=pl.BlockSpec((_BR, _DC), lambda i: (i, 0)),
        out_shape=jax.ShapeDtypeStruct((_NACC, _DC), jnp.float32),
    )(parts, nd, ns, W, b, W2p)


def _tc_final(parts, nd, b2p):
    def body(p_ref, nd_ref, b_ref, o_ref):
        o_ref[...] = (p_ref[0] + p_ref[1]) * nd_ref[...] + b_ref[...]

    return pl.pallas_call(
        body,
        grid=(_GRID,),
        in_specs=[
            pl.BlockSpec((_NC, _BR, _DC), lambda i: (0, i, 0)),
            pl.BlockSpec((_BR, 1), lambda i: (i, 0)),
            pl.BlockSpec((1, _DC), lambda i: (0, 0)),
        ],
        out_specs=pl.BlockSpec((_BR, _DC), lambda i: (i, 0)),
        out_shape=jax.ShapeDtypeStruct((_NACC, _DC), jnp.float32),
    )(parts, nd, b2p)


def kernel(x, edge_index, W0, b0, W1, b1, W2, b2):
    src = edge_index[0]
    dst = edge_index[1]
    padi = jnp.full((_EPAD - _E,), _N, jnp.int32)
    srcp = jnp.concatenate([src, padi])
    dstp = jnp.concatenate([dst, padi])
    degidx = jnp.concatenate([srcp, dstp + _NACC])
    ones16 = jnp.ones((_C, 16), jnp.float32)
    zdeg = jnp.zeros((_L2, 16), jnp.float32)
    z128 = jnp.zeros((_NACC, _D), jnp.float32)
    z48 = jnp.zeros((_NACC, _DC), jnp.float32)

    degparts = _deg_kernel(degidx, ones16, zdeg).reshape(_NC, 2, _NACC, 16)
    xpad = jnp.pad(x, ((0, _NACC - _N), (0, 0)))
    xs, ns, nd = _tc_norm_scale(xpad, degparts)

    agg128 = _make_agg(_D)
    p0 = agg128(xs, srcp, dstp, z128).reshape(_NC, _NACC, _D)
    h1 = _tc_layer(p0, nd, ns, W0, b0.reshape(1, _D))
    p1 = agg128(h1, srcp, dstp, z128).reshape(_NC, _NACC, _D)
    W2p = jnp.pad(W2, ((0, 0), (0, _DC - _NCLS)))
    z2 = _tc_layer_premat(p1, nd, ns, W1, b1.reshape(1, _D), W2p)
    p2 = _make_agg(_DC)(z2, srcp, dstp, z48).reshape(_NC, _NACC, _DC)
    b2p = jnp.pad(b2, (0, _DC - _NCLS)).reshape(1, _DC)
    outp = _tc_final(p2, nd, b2p)
    return outp[:_N, :_NCLS]


# speedup vs baseline: 3.2208x; 3.2208x over previous
import functools

import jax
import jax.numpy as jnp
from jax import lax
from jax.experimental import pallas as pl
from jax.experimental.pallas import tpu as pltpu
from jax.experimental.pallas import tpu_sc as plsc

_N = 10000
_NACC = 10240
_D = 128
_NCLS = 40
_DC = 48
_E = 320000
_EPAD = 327680
_NC = 2
_NS = 16
_NW = _NC * _NS
_EPT = _EPAD // _NW
_C = 128
_NCH = _EPT // _C
_RPT = _NACC // _NS
_L2 = 2 * _NACC
_EPT2 = 2 * _EPT
_NCH2 = _EPT2 // _C
_RPT2 = _L2 // _NS
_BR = 512
_GRID = _NACC // _BR


def _sc_mesh():
    return plsc.VectorSubcoreMesh(core_axis_name="c", subcore_axis_name="s")


@functools.lru_cache(maxsize=None)
def _make_agg(D):
    @functools.partial(
        pl.kernel,
        out_type=jax.ShapeDtypeStruct((_NC * _NACC, D), jnp.float32),
        mesh=_sc_mesh(),
        compiler_params=pltpu.CompilerParams(use_tc_tiling_on_sc=(D % 128 == 0)),
        scratch_types=[
            pltpu.VMEM((_C,), jnp.int32),
            pltpu.VMEM((_C,), jnp.int32),
            pltpu.VMEM((_C, D), jnp.float32),
            pltpu.VMEM_SHARED((_NACC, D), jnp.float32),
            pltpu.SemaphoreType.DMA,
        ],
    )
    def agg(table, srcp, dstp, zeros, out, sidx, didx, rows, acc, sem):
        c = lax.axis_index("c")
        s = lax.axis_index("s")
        wid = c * _NS + s
        rbase = s * _RPT
        pltpu.sync_copy(zeros.at[pl.ds(rbase, _RPT)], acc.at[pl.ds(rbase, _RPT)])
        plsc.subcore_barrier()
        ebase = wid * _EPT

        def body(j, carry):
            off = ebase + j * _C
            pltpu.sync_copy(srcp.at[pl.ds(off, _C)], sidx)
            pltpu.sync_copy(dstp.at[pl.ds(off, _C)], didx)
            pltpu.async_copy(table.at[sidx], rows, sem).wait()
            pltpu.sync_copy(rows, acc.at[didx], add=True)
            return carry

        lax.fori_loop(0, _NCH, body, 0)
        plsc.subcore_barrier()
        pltpu.sync_copy(acc.at[pl.ds(rbase, _RPT)],
                        out.at[pl.ds(c * _NACC + rbase, _RPT)])

    return agg


@functools.partial(
    pl.kernel,
    out_type=jax.ShapeDtypeStruct((_NC * _L2, 16), jnp.float32),
    mesh=_sc_mesh(),
    compiler_params=pltpu.CompilerParams(use_tc_tiling_on_sc=False),
    scratch_types=[
        pltpu.VMEM((_C,), jnp.int32),
        pltpu.VMEM((_C, 16), jnp.float32),
        pltpu.VMEM_SHARED((_L2, 16), jnp.float32),
    ],
)
def _deg_kernel(idxs, ones, zeros, out, idxbuf, ones_v, acc):
    c = lax.axis_index("c")
    s = lax.axis_index("s")
    wid = c * _NS + s
    rbase = s * _RPT2
    pltpu.sync_copy(ones, ones_v)
    pltpu.sync_copy(zeros.at[pl.ds(rbase, _RPT2)], acc.at[pl.ds(rbase, _RPT2)])
    plsc.subcore_barrier()
    base = wid * _EPT2

    def body(j, carry):
        pltpu.sync_copy(idxs.at[pl.ds(base + j * _C, _C)], idxbuf)
        pltpu.sync_copy(ones_v, acc.at[idxbuf], add=True)
        return carry

    lax.fori_loop(0, _NCH2, body, 0)
    plsc.subcore_barrier()
    pltpu.sync_copy(acc.at[pl.ds(rbase, _RPT2)],
                    out.at[pl.ds(c * _L2 + rbase, _RPT2)])


def _tc_norm_scale(xpad, degparts):
    def body(x_ref, dp_ref, xs_ref, ns_ref, nd_ref):
        d = dp_ref[0] + dp_ref[1]
        ns = lax.rsqrt(jnp.maximum(d[0, :, 0:1], 1.0))
        nd = lax.rsqrt(jnp.maximum(d[1, :, 0:1], 1.0))
        xs_ref[...] = x_ref[...] * ns
        ns_ref[...] = ns
        nd_ref[...] = nd

    return pl.pallas_call(
        body,
        grid=(_GRID,),
        in_specs=[
            pl.BlockSpec((_BR, _D), lambda i: (i, 0)),
            pl.BlockSpec((_NC, 2, _BR, 16), lambda i: (0, 0, i, 0)),
        ],
        out_specs=[
            pl.BlockSpec((_BR, _D), lambda i: (i, 0)),
            pl.BlockSpec((_BR, 1), lambda i: (i, 0)),
            pl.BlockSpec((_BR, 1), lambda i: (i, 0)),
        ],
        out_shape=[
            jax.ShapeDtypeStruct((_NACC, _D), jnp.float32),
            jax.ShapeDtypeStruct((_NACC, 1), jnp.float32),
            jax.ShapeDtypeStruct((_NACC, 1), jnp.float32),
        ],
    )(xpad, degparts)


def _tc_layer(parts, nd, ns, W, b):
    def body(p_ref, nd_ref, ns_ref, w_ref, b_ref, o_ref):
        h = (p_ref[0] + p_ref[1]) * nd_ref[...]
        z = jnp.dot(h, w_ref[...], preferred_element_type=jnp.float32) + b_ref[...]
        o_ref[...] = jnp.where(z > 0.0, z, jnp.exp(z) - 1.0) * ns_ref[...]

    return pl.pallas_call(
        body,
        grid=(_GRID,),
        in_specs=[
            pl.BlockSpec((_NC, _BR, _D), lambda i: (0, i, 0)),
            pl.BlockSpec((_BR, 1), lambda i: (i, 0)),
            pl.BlockSpec((_BR, 1), lambda i: (i, 0)),
            pl.BlockSpec((_D, _D), lambda i: (0, 0)),
            pl.BlockSpec((1, _D), lambda i: (0, 0)),
        ],
        out_specs=pl.BlockSpec((_BR, _D), lambda i: (i, 0)),
        out_shape=jax.ShapeDtypeStruct((_NACC, _D), jnp.float32),
    )(parts, nd, ns, W, b)


def _tc_layer_premat(parts, nd, ns, W, b, W2p):
    def body(p_ref, nd_ref, ns_ref, w_ref, b_ref, w2_ref, o_ref):
        h = (p_ref[0] + p_ref[1]) * nd_ref[...]
        z = jnp.dot(h, w_ref[...], preferred_element_type=jnp.float32) + b_ref[...]
        h2 = jnp.where(z > 0.0, z, jnp.exp(z) - 1.0) * ns_ref[...]
        o_ref[...] = jnp.dot(h2, w2_ref[...], preferred_element_type=jnp.float32)

    return pl.pallas_call(
        body,
        grid=(_GRID,),
        in_specs=[
            pl.BlockSpec((_NC, _BR, _D), lambda i: (0, i, 0)),
            pl.BlockSpec((_BR, 1), lambda i: (i, 0)),
            pl.BlockSpec((_BR, 1), lambda i: (i, 0)),
            pl.BlockSpec((_D, _D), lambda i: (0, 0)),
            pl.BlockSpec((1, _D), lambda i: (0, 0)),
            pl.BlockSpec((_D, _DC), lambda i: (0, 0)),
        ],
        out_specs=pl.BlockSpec((_BR, _DC), lambda i: (i, 0)),
        out_shape=jax.ShapeDtypeStruct((_NACC, _DC), jnp.float32),
    )(parts, nd, ns, W, b, W2p)


def _tc_final(parts, nd, b2p):
    def body(p_ref, nd_ref, b_ref, o_ref):
        o_ref[...] = (p_ref[0] + p_ref[1]) * nd_ref[...] + b_ref[...]

    return pl.pallas_call(
        body,
        grid=(_GRID,),
        in_specs=[
            pl.BlockSpec((_NC, _BR, _DC), lambda i: (0, i, 0)),
            pl.BlockSpec((_BR, 1), lambda i: (i, 0)),
            pl.BlockSpec((1, _DC), lambda i: (0, 0)),
        ],
        out_specs=pl.BlockSpec((_BR, _DC), lambda i: (i, 0)),
        out_shape=jax.ShapeDtypeStruct((_NACC, _DC), jnp.float32),
    )(parts, nd, b2p)


def kernel(x, edge_index, W0, b0, W1, b1, W2, b2):
    src = edge_index[0]
    dst = edge_index[1]
    padi = jnp.full((_EPAD - _E,), _N, jnp.int32)
    srcp = jnp.concatenate([src, padi])
    dstp = jnp.concatenate([dst, padi])
    degidx = jnp.concatenate([srcp, dstp + _NACC])
    ones16 = jnp.ones((_C, 16), jnp.float32)
    zdeg = jnp.zeros((_L2, 16), jnp.float32)
    z128 = jnp.zeros((_NACC, _D), jnp.float32)
    z48 = jnp.zeros((_NACC, _DC), jnp.float32)

    degparts = _deg_kernel(degidx, ones16, zdeg).reshape(_NC, 2, _NACC, 16)
    xpad = jnp.pad(x, ((0, _NACC - _N), (0, 0)))
    xs, ns, nd = _tc_norm_scale(xpad, degparts)

    agg128 = _make_agg(_D)
    p0 = agg128(xs, srcp, dstp, z128).reshape(_NC, _NACC, _D)
    h1 = _tc_layer(p0, nd, ns, W0, b0.reshape(1, _D))
    p1 = agg128(h1, srcp, dstp, z128).reshape(_NC, _NACC, _D)
    W2p = jnp.pad(W2, ((0, 0), (0, _DC - _NCLS)))
    z2 = _tc_layer_premat(p1, nd, ns, W1, b1.reshape(1, _D), W2p)
    p2 = _make_agg(_DC)(z2, srcp, dstp, z48).reshape(_NC, _NACC, _DC)
    b2p = jnp.pad(b2, (0, _DC - _NCLS)).reshape(1, _DC)
    outp = _tc_final(p2, nd, b2p)
    return outp[:_N, :_NCLS]


# speedup vs baseline: 3.6658x; 1.1382x over previous
import functools

import jax
import jax.numpy as jnp
from jax import lax
from jax.experimental import pallas as pl
from jax.experimental.pallas import tpu as pltpu
from jax.experimental.pallas import tpu_sc as plsc

_N = 10000
_NACC = 10240
_D = 128
_NCLS = 40
_DC = 48
_E = 320000
_EPAD = 327680
_NC = 2
_NS = 16
_NW = _NC * _NS
_EPT = _EPAD // _NW
_C = 128
_NCH = _EPT // _C
_RPT = _NACC // _NS
_L2 = 2 * _NACC
_EPT2 = 2 * _EPT
_NCH2 = _EPT2 // _C
_RPT2 = _L2 // _NS
_BR = 512
_GRID = _NACC // _BR


def _sc_mesh():
    return plsc.VectorSubcoreMesh(core_axis_name="c", subcore_axis_name="s")


_NBUF = 4


@functools.lru_cache(maxsize=None)
def _make_agg(D):
    G = _NCH // _NBUF

    @functools.partial(
        pl.kernel,
        out_type=jax.ShapeDtypeStruct((_NC * _NACC, D), jnp.float32),
        mesh=_sc_mesh(),
        compiler_params=pltpu.CompilerParams(use_tc_tiling_on_sc=(D % 128 == 0)),
        scratch_types=(
            [pltpu.VMEM((_NCH, _C), jnp.int32)] * 2
            + [pltpu.VMEM((_C, D), jnp.float32)] * _NBUF
            + [pltpu.VMEM_SHARED((_NACC, D), jnp.float32)]
            + [pltpu.SemaphoreType.DMA] * (2 * _NBUF)
        ),
    )
    def agg(table, srcp2, dstp2, zeros, out, sidx, didx, *scr):
        rows = scr[:_NBUF]
        acc = scr[_NBUF]
        gsem = scr[_NBUF + 1:2 * _NBUF + 1]
        ssem = scr[2 * _NBUF + 1:]
        c = lax.axis_index("c")
        s = lax.axis_index("s")
        wid = c * _NS + s
        rbase = s * _RPT
        pltpu.sync_copy(zeros.at[pl.ds(rbase, _RPT)], acc.at[pl.ds(rbase, _RPT)])
        pltpu.sync_copy(srcp2.at[pl.ds(wid * _NCH, _NCH)], sidx)
        pltpu.sync_copy(dstp2.at[pl.ds(wid * _NCH, _NCH)], didx)
        plsc.subcore_barrier()

        def group(g, carry):
            jb = g * _NBUF
            for b in range(_NBUF):
                pltpu.async_copy(table.at[sidx.at[jb + b]], rows[b], gsem[b])
            for b in range(_NBUF):
                pltpu.make_async_copy(
                    table.at[sidx.at[jb + b]], rows[b], gsem[b]).wait()
                pltpu.async_copy(rows[b], acc.at[didx.at[jb + b]],
                                 ssem[b], add=True)
            for b in range(_NBUF):
                pltpu.make_async_copy(
                    rows[b], acc.at[didx.at[jb + b]], ssem[b]).wait()
            return carry

        lax.fori_loop(0, G, group, 0)
        plsc.subcore_barrier()
        pltpu.sync_copy(acc.at[pl.ds(rbase, _RPT)],
                        out.at[pl.ds(c * _NACC + rbase, _RPT)])

    return agg


_DEGK = 8


@functools.partial(
    pl.kernel,
    out_type=jax.ShapeDtypeStruct((_NC * _L2, 16), jnp.float32),
    mesh=_sc_mesh(),
    compiler_params=pltpu.CompilerParams(use_tc_tiling_on_sc=False),
    scratch_types=[
        pltpu.VMEM((_NCH2, _C), jnp.int32),
        pltpu.VMEM((_C, 16), jnp.float32),
        pltpu.VMEM_SHARED((_L2, 16), jnp.float32),
        pltpu.SemaphoreType.DMA,
        pltpu.SemaphoreType.DMA,
    ],
)
def _deg_kernel(idxs2, ones, zeros, out, idxbuf, ones_v, acc, sem0, sem1):
    sems = (sem0, sem1)
    c = lax.axis_index("c")
    s = lax.axis_index("s")
    wid = c * _NS + s
    rbase = s * _RPT2
    pltpu.sync_copy(ones, ones_v)
    pltpu.sync_copy(zeros.at[pl.ds(rbase, _RPT2)], acc.at[pl.ds(rbase, _RPT2)])
    pltpu.sync_copy(idxs2.at[pl.ds(wid * _NCH2, _NCH2)], idxbuf)
    plsc.subcore_barrier()
    NG = _NCH2 // _DEGK

    def dgroup(g, carry):
        for k in range(_DEGK):
            pltpu.async_copy(ones_v, acc.at[idxbuf.at[g * _DEGK + k]],
                             sems[k % 2], add=True)
        for k in range(_DEGK):
            pltpu.make_async_copy(
                ones_v, acc.at[idxbuf.at[g * _DEGK + k]], sems[k % 2]).wait()
        return carry

    lax.fori_loop(0, NG, dgroup, 0)
    plsc.subcore_barrier()
    pltpu.sync_copy(acc.at[pl.ds(rbase, _RPT2)],
                    out.at[pl.ds(c * _L2 + rbase, _RPT2)])


def _tc_norm_scale(xpad, degparts):
    def body(x_ref, dp_ref, xs_ref, ns_ref, nd_ref):
        d = dp_ref[0] + dp_ref[1]
        ns = lax.rsqrt(jnp.maximum(d[0, :, 0:1], 1.0))
        nd = lax.rsqrt(jnp.maximum(d[1, :, 0:1], 1.0))
        xs_ref[...] = x_ref[...] * ns
        ns_ref[...] = ns
        nd_ref[...] = nd

    return pl.pallas_call(
        body,
        grid=(_GRID,),
        in_specs=[
            pl.BlockSpec((_BR, _D), lambda i: (i, 0)),
            pl.BlockSpec((_NC, 2, _BR, 16), lambda i: (0, 0, i, 0)),
        ],
        out_specs=[
            pl.BlockSpec((_BR, _D), lambda i: (i, 0)),
            pl.BlockSpec((_BR, 1), lambda i: (i, 0)),
            pl.BlockSpec((_BR, 1), lambda i: (i, 0)),
        ],
        out_shape=[
            jax.ShapeDtypeStruct((_NACC, _D), jnp.float32),
            jax.ShapeDtypeStruct((_NACC, 1), jnp.float32),
            jax.ShapeDtypeStruct((_NACC, 1), jnp.float32),
        ],
    )(xpad, degparts)


def _tc_layer(pa, pb, nd, ns, W, b):
    def body(pa_ref, pb_ref, nd_ref, ns_ref, w_ref, b_ref, o_ref):
        h = jnp.concatenate(
            [pa_ref[0] + pa_ref[1], pb_ref[0] + pb_ref[1]], axis=-1
        ) * nd_ref[...]
        z = jnp.dot(h, w_ref[...], preferred_element_type=jnp.float32) + b_ref[...]
        o_ref[...] = jnp.where(z > 0.0, z, jnp.exp(z) - 1.0) * ns_ref[...]

    return pl.pallas_call(
        body,
        grid=(_GRID,),
        in_specs=[
            pl.BlockSpec((_NC, _BR, _D // 2), lambda i: (0, i, 0)),
            pl.BlockSpec((_NC, _BR, _D // 2), lambda i: (0, i, 0)),
            pl.BlockSpec((_BR, 1), lambda i: (i, 0)),
            pl.BlockSpec((_BR, 1), lambda i: (i, 0)),
            pl.BlockSpec((_D, _D), lambda i: (0, 0)),
            pl.BlockSpec((1, _D), lambda i: (0, 0)),
        ],
        out_specs=pl.BlockSpec((_BR, _D), lambda i: (i, 0)),
        out_shape=jax.ShapeDtypeStruct((_NACC, _D), jnp.float32),
    )(pa, pb, nd, ns, W, b)


def _tc_layer_premat(pa, pb, nd, ns, W, b, W2p):
    def body(pa_ref, pb_ref, nd_ref, ns_ref, w_ref, b_ref, w2_ref, o_ref):
        h = jnp.concatenate(
            [pa_ref[0] + pa_ref[1], pb_ref[0] + pb_ref[1]], axis=-1
        ) * nd_ref[...]
        z = jnp.dot(h, w_ref[...], preferred_element_type=jnp.float32) + b_ref[...]
        h2 = jnp.where(z > 0.0, z, jnp.exp(z) - 1.0) * ns_ref[...]
        o_ref[...] = jnp.dot(h2, w2_ref[...], preferred_element_type=jnp.float32)

    return pl.pallas_call(
        body,
        grid=(_GRID,),
        in_specs=[
            pl.BlockSpec((_NC, _BR, _D // 2), lambda i: (0, i, 0)),
            pl.BlockSpec((_NC, _BR, _D // 2), lambda i: (0, i, 0)),
            pl.BlockSpec((_BR, 1), lambda i: (i, 0)),
            pl.BlockSpec((_BR, 1), lambda i: (i, 0)),
            pl.BlockSpec((_D, _D), lambda i: (0, 0)),
            pl.BlockSpec((1, _D), lambda i: (0, 0)),
            pl.BlockSpec((_D, _DC), lambda i: (0, 0)),
        ],
        out_specs=pl.BlockSpec((_BR, _DC), lambda i: (i, 0)),
        out_shape=jax.ShapeDtypeStruct((_NACC, _DC), jnp.float32),
    )(pa, pb, nd, ns, W, b, W2p)


def _tc_final(parts, nd, b2p):
    def body(p_ref, nd_ref, b_ref, o_ref):
        o_ref[...] = (p_ref[0] + p_ref[1]) * nd_ref[...] + b_ref[...]

    return pl.pallas_call(
        body,
        grid=(_GRID,),
        in_specs=[
            pl.BlockSpec((_NC, _BR, _DC), lambda i: (0, i, 0)),
            pl.BlockSpec((_BR, 1), lambda i: (i, 0)),
            pl.BlockSpec((1, _DC), lambda i: (0, 0)),
        ],
        out_specs=pl.BlockSpec((_BR, _DC), lambda i: (i, 0)),
        out_shape=jax.ShapeDtypeStruct((_NACC, _DC), jnp.float32),
    )(parts, nd, b2p)


def kernel(x, edge_index, W0, b0, W1, b1, W2, b2):
    src = edge_index[0]
    dst = edge_index[1]
    padi = jnp.full((_EPAD - _E,), _N, jnp.int32)
    srcp = jnp.concatenate([src, padi])
    dstp = jnp.concatenate([dst, padi])
    degidx = jnp.concatenate([srcp, dstp + _NACC])
    ones16 = jnp.ones((_C, 16), jnp.float32)
    zdeg = jnp.zeros((_L2, 16), jnp.float32)
    z64 = jnp.zeros((_NACC, _D // 2), jnp.float32)
    z48 = jnp.zeros((_NACC, _DC), jnp.float32)

    degidx2 = degidx.reshape(_NW * _NCH2, _C)
    srcp2 = srcp.reshape(_NW * _NCH, _C)
    dstp2 = dstp.reshape(_NW * _NCH, _C)

    degparts = _deg_kernel(degidx2, ones16, zdeg).reshape(_NC, 2, _NACC, 16)
    xpad = jnp.pad(x, ((0, _NACC - _N), (0, 0)))
    xs, ns, nd = _tc_norm_scale(xpad, degparts)

    agg64 = _make_agg(_D // 2)
    H = _D // 2
    p0a = agg64(xs[:, :H], srcp2, dstp2, z64).reshape(_NC, _NACC, H)
    p0b = agg64(xs[:, H:], srcp2, dstp2, z64).reshape(_NC, _NACC, H)
    h1 = _tc_layer(p0a, p0b, nd, ns, W0, b0.reshape(1, _D))
    p1a = agg64(h1[:, :H], srcp2, dstp2, z64).reshape(_NC, _NACC, H)
    p1b = agg64(h1[:, H:], srcp2, dstp2, z64).reshape(_NC, _NACC, H)
    W2p = jnp.pad(W2, ((0, 0), (0, _DC - _NCLS)))
    z2 = _tc_layer_premat(p1a, p1b, nd, ns, W1, b1.reshape(1, _D), W2p)
    p2 = _make_agg(_DC)(z2, srcp2, dstp2, z48).reshape(_NC, _NACC, _DC)
    b2p = jnp.pad(b2, (0, _DC - _NCLS)).reshape(1, _DC)
    outp = _tc_final(p2, nd, b2p)
    return outp[:_N, :_NCLS]


# speedup vs baseline: 9.2889x; 2.5339x over previous
import functools

import jax
import jax.numpy as jnp
from jax import lax
from jax.experimental import pallas as pl
from jax.experimental.pallas import tpu as pltpu
from jax.experimental.pallas import tpu_sc as plsc

_N = 10000
_NACC = 10240
_D = 128
_NCLS = 40
_DC = 48
_E = 320000
_EPAD = 327680
_NC = 2
_NS = 16
_NW = _NC * _NS
_EPT = _EPAD // _NW
_C = 128
_NCH = _EPT // _C
_RPT = _NACC // _NS
_L2 = 2 * _NACC
_EPT2 = 2 * _EPT
_NCH2 = _EPT2 // _C
_RPT2 = _L2 // _NS
_BR = 512
_GRID = _NACC // _BR


def _sc_mesh():
    return plsc.VectorSubcoreMesh(core_axis_name="c", subcore_axis_name="s")


_NBUF = 4


@functools.lru_cache(maxsize=None)
def _make_agg(D):
    G = _NCH // _NBUF

    @functools.partial(
        pl.kernel,
        out_type=jax.ShapeDtypeStruct((_NC * _NACC, D), jnp.float32),
        mesh=_sc_mesh(),
        compiler_params=pltpu.CompilerParams(use_tc_tiling_on_sc=(D % 128 == 0)),
        scratch_types=(
            [pltpu.VMEM((_NCH, _C), jnp.int32)] * 2
            + [pltpu.VMEM((_C, D), jnp.float32)] * _NBUF
            + [pltpu.VMEM_SHARED((_NACC, D), jnp.float32)]
            + [pltpu.SemaphoreType.DMA] * (2 * _NBUF)
        ),
    )
    def agg(table, srcp2, dstp2, zeros, out, sidx, didx, *scr):
        rows = scr[:_NBUF]
        acc = scr[_NBUF]
        gsem = scr[_NBUF + 1:2 * _NBUF + 1]
        ssem = scr[2 * _NBUF + 1:]
        c = lax.axis_index("c")
        s = lax.axis_index("s")
        wid = c * _NS + s
        rbase = s * _RPT
        pltpu.sync_copy(zeros.at[pl.ds(rbase, _RPT)], acc.at[pl.ds(rbase, _RPT)])
        pltpu.sync_copy(srcp2.at[pl.ds(wid * _NCH, _NCH)], sidx)
        pltpu.sync_copy(dstp2.at[pl.ds(wid * _NCH, _NCH)], didx)
        plsc.subcore_barrier()

        def group(g, carry):
            jb = g * _NBUF
            for b in range(_NBUF):
                pltpu.async_copy(table.at[sidx.at[jb + b]], rows[b], gsem[b])
            for b in range(_NBUF):
                pltpu.make_async_copy(
                    table.at[sidx.at[jb + b]], rows[b], gsem[b]).wait()
                pltpu.async_copy(rows[b], acc.at[didx.at[jb + b]],
                                 ssem[b], add=True)
            for b in range(_NBUF):
                pltpu.make_async_copy(
                    rows[b], acc.at[didx.at[jb + b]], ssem[b]).wait()
            return carry

        lax.fori_loop(0, G, group, 0)
        plsc.subcore_barrier()
        pltpu.sync_copy(acc.at[pl.ds(rbase, _RPT)],
                        out.at[pl.ds(c * _NACC + rbase, _RPT)])

    return agg


_DEGK = 8


@functools.partial(
    pl.kernel,
    out_type=jax.ShapeDtypeStruct((_NC * _L2, 16), jnp.float32),
    mesh=_sc_mesh(),
    compiler_params=pltpu.CompilerParams(use_tc_tiling_on_sc=False),
    scratch_types=[
        pltpu.VMEM((_NCH2, _C), jnp.int32),
        pltpu.VMEM((_C, 16), jnp.float32),
        pltpu.VMEM_SHARED((_L2, 16), jnp.float32),
        pltpu.SemaphoreType.DMA,
        pltpu.SemaphoreType.DMA,
    ],
)
def _deg_kernel(idxs2, ones, zeros, out, idxbuf, ones_v, acc, sem0, sem1):
    sems = (sem0, sem1)
    c = lax.axis_index("c")
    s = lax.axis_index("s")
    wid = c * _NS + s
    rbase = s * _RPT2
    pltpu.sync_copy(ones, ones_v)
    pltpu.sync_copy(zeros.at[pl.ds(rbase, _RPT2)], acc.at[pl.ds(rbase, _RPT2)])
    pltpu.sync_copy(idxs2.at[pl.ds(wid * _NCH2, _NCH2)], idxbuf)
    plsc.subcore_barrier()
    NG = _NCH2 // _DEGK

    def dgroup(g, carry):
        for k in range(_DEGK):
            pltpu.async_copy(ones_v, acc.at[idxbuf.at[g * _DEGK + k]],
                             sems[k % 2], add=True)
        for k in range(_DEGK):
            pltpu.make_async_copy(
                ones_v, acc.at[idxbuf.at[g * _DEGK + k]], sems[k % 2]).wait()
        return carry

    lax.fori_loop(0, NG, dgroup, 0)
    plsc.subcore_barrier()
    pltpu.sync_copy(acc.at[pl.ds(rbase, _RPT2)],
                    out.at[pl.ds(c * _L2 + rbase, _RPT2)])


def _tc_norm_scale(xpad, degparts):
    def body(x_ref, dp_ref, xs_ref, ns_ref, nd_ref):
        d = dp_ref[0] + dp_ref[1]
        ns = lax.rsqrt(jnp.maximum(d[0, :, 0:1], 1.0))
        nd = lax.rsqrt(jnp.maximum(d[1, :, 0:1], 1.0))
        xs_ref[...] = x_ref[...] * ns
        ns_ref[...] = ns
        nd_ref[...] = nd

    return pl.pallas_call(
        body,
        grid=(_GRID,),
        in_specs=[
            pl.BlockSpec((_BR, _D), lambda i: (i, 0)),
            pl.BlockSpec((_NC, 2, _BR, 16), lambda i: (0, 0, i, 0)),
        ],
        out_specs=[
            pl.BlockSpec((_BR, _D), lambda i: (i, 0)),
            pl.BlockSpec((_BR, 1), lambda i: (i, 0)),
            pl.BlockSpec((_BR, 1), lambda i: (i, 0)),
        ],
        out_shape=[
            jax.ShapeDtypeStruct((_NACC, _D), jnp.float32),
            jax.ShapeDtypeStruct((_NACC, 1), jnp.float32),
            jax.ShapeDtypeStruct((_NACC, 1), jnp.float32),
        ],
    )(xpad, degparts)


def _tc_layer(pa, pb, nd, ns, W, b):
    def body(pa_ref, pb_ref, nd_ref, ns_ref, w_ref, b_ref, o_ref):
        h = jnp.concatenate(
            [pa_ref[0] + pa_ref[1], pb_ref[0] + pb_ref[1]], axis=-1
        ) * nd_ref[...]
        z = jnp.dot(h, w_ref[...], preferred_element_type=jnp.float32) + b_ref[...]
        o_ref[...] = jnp.where(z > 0.0, z, jnp.exp(z) - 1.0) * ns_ref[...]

    return pl.pallas_call(
        body,
        grid=(_GRID,),
        in_specs=[
            pl.BlockSpec((_NC, _BR, _D // 2), lambda i: (0, i, 0)),
            pl.BlockSpec((_NC, _BR, _D // 2), lambda i: (0, i, 0)),
            pl.BlockSpec((_BR, 1), lambda i: (i, 0)),
            pl.BlockSpec((_BR, 1), lambda i: (i, 0)),
            pl.BlockSpec((_D, _D), lambda i: (0, 0)),
            pl.BlockSpec((1, _D), lambda i: (0, 0)),
        ],
        out_specs=pl.BlockSpec((_BR, _D), lambda i: (i, 0)),
        out_shape=jax.ShapeDtypeStruct((_NACC, _D), jnp.float32),
    )(pa, pb, nd, ns, W, b)


def _tc_layer_premat(pa, pb, nd, ns, W, b, W2p):
    def body(pa_ref, pb_ref, nd_ref, ns_ref, w_ref, b_ref, w2_ref, o_ref):
        h = jnp.concatenate(
            [pa_ref[0] + pa_ref[1], pb_ref[0] + pb_ref[1]], axis=-1
        ) * nd_ref[...]
        z = jnp.dot(h, w_ref[...], preferred_element_type=jnp.float32) + b_ref[...]
        h2 = jnp.where(z > 0.0, z, jnp.exp(z) - 1.0) * ns_ref[...]
        o_ref[...] = jnp.dot(h2, w2_ref[...], preferred_element_type=jnp.float32)

    return pl.pallas_call(
        body,
        grid=(_GRID,),
        in_specs=[
            pl.BlockSpec((_NC, _BR, _D // 2), lambda i: (0, i, 0)),
            pl.BlockSpec((_NC, _BR, _D // 2), lambda i: (0, i, 0)),
            pl.BlockSpec((_BR, 1), lambda i: (i, 0)),
            pl.BlockSpec((_BR, 1), lambda i: (i, 0)),
            pl.BlockSpec((_D, _D), lambda i: (0, 0)),
            pl.BlockSpec((1, _D), lambda i: (0, 0)),
            pl.BlockSpec((_D, _DC), lambda i: (0, 0)),
        ],
        out_specs=pl.BlockSpec((_BR, _DC), lambda i: (i, 0)),
        out_shape=jax.ShapeDtypeStruct((_NACC, _DC), jnp.float32),
    )(pa, pb, nd, ns, W, b, W2p)


def _tc_final(parts, nd, b2p):
    def body(p_ref, nd_ref, b_ref, o_ref):
        o_ref[...] = (p_ref[0] + p_ref[1]) * nd_ref[...] + b_ref[...]

    return pl.pallas_call(
        body,
        grid=(_GRID,),
        in_specs=[
            pl.BlockSpec((_NC, _BR, _DC), lambda i: (0, i, 0)),
            pl.BlockSpec((_BR, 1), lambda i: (i, 0)),
            pl.BlockSpec((1, _DC), lambda i: (0, 0)),
        ],
        out_specs=pl.BlockSpec((_BR, _DC), lambda i: (i, 0)),
        out_shape=jax.ShapeDtypeStruct((_NACC, _DC), jnp.float32),
    )(parts, nd, b2p)


def kernel(x, edge_index, W0, b0, W1, b1, W2, b2):
    src = edge_index[0]
    dst = edge_index[1]
    padi = _N + (jnp.arange(_EPAD - _E, dtype=jnp.int32) % (_NACC - _N))
    srcp = jnp.concatenate([src, padi])
    dstp = jnp.concatenate([dst, padi])
    degidx = jnp.concatenate([srcp, dstp + _NACC])
    ones16 = jnp.ones((_C, 16), jnp.float32)
    zdeg = jnp.zeros((_L2, 16), jnp.float32)
    z64 = jnp.zeros((_NACC, _D // 2), jnp.float32)
    z48 = jnp.zeros((_NACC, _DC), jnp.float32)

    degidx2 = degidx.reshape(_NW * _NCH2, _C)
    srcp2 = srcp.reshape(_NW * _NCH, _C)
    dstp2 = dstp.reshape(_NW * _NCH, _C)

    degparts = _deg_kernel(degidx2, ones16, zdeg).reshape(_NC, 2, _NACC, 16)
    xpad = jnp.pad(x, ((0, _NACC - _N), (0, 0)))
    xs, ns, nd = _tc_norm_scale(xpad, degparts)

    agg64 = _make_agg(_D // 2)
    H = _D // 2
    p0a = agg64(xs[:, :H], srcp2, dstp2, z64).reshape(_NC, _NACC, H)
    p0b = agg64(xs[:, H:], srcp2, dstp2, z64).reshape(_NC, _NACC, H)
    h1 = _tc_layer(p0a, p0b, nd, ns, W0, b0.reshape(1, _D))
    p1a = agg64(h1[:, :H], srcp2, dstp2, z64).reshape(_NC, _NACC, H)
    p1b = agg64(h1[:, H:], srcp2, dstp2, z64).reshape(_NC, _NACC, H)
    W2p = jnp.pad(W2, ((0, 0), (0, _DC - _NCLS)))
    z2 = _tc_layer_premat(p1a, p1b, nd, ns, W1, b1.reshape(1, _D), W2p)
    p2 = _make_agg(_DC)(z2, srcp2, dstp2, z48).reshape(_NC, _NACC, _DC)
    b2p = jnp.pad(b2, (0, _DC - _NCLS)).reshape(1, _DC)
    outp = _tc_final(p2, nd, b2p)
    return outp[:_N, :_NCLS]


# speedup vs baseline: 9.4741x; 1.0199x over previous
import functools

import jax
import jax.numpy as jnp
from jax import lax
from jax.experimental import pallas as pl
from jax.experimental.pallas import tpu as pltpu
from jax.experimental.pallas import tpu_sc as plsc

_N = 10000
_NACC = 10240
_D = 128
_NCLS = 40
_DC = 48
_E = 320000
_EPAD = 327680
_NC = 2
_NS = 16
_NW = _NC * _NS
_EPT = _EPAD // _NW
_C = 128
_NCH = _EPT // _C
_RPT = _NACC // _NS
_L2 = 2 * _NACC
_EPT2 = 2 * _EPT
_NCH2 = _EPT2 // _C
_RPT2 = _L2 // _NS
_BR = 512
_GRID = _NACC // _BR


def _sc_mesh():
    return plsc.VectorSubcoreMesh(core_axis_name="c", subcore_axis_name="s")


_NBUF = 4


@functools.lru_cache(maxsize=None)
def _make_agg(D):
    G = _NCH // _NBUF

    @functools.partial(
        pl.kernel,
        out_type=jax.ShapeDtypeStruct((_NC * _NACC, D), jnp.float32),
        mesh=_sc_mesh(),
        compiler_params=pltpu.CompilerParams(use_tc_tiling_on_sc=(D % 128 == 0)),
        scratch_types=(
            [pltpu.VMEM((_NCH, _C), jnp.int32)] * 2
            + [pltpu.VMEM((_C, D), jnp.float32)] * _NBUF
            + [pltpu.VMEM_SHARED((_NACC, D), jnp.float32)]
            + [pltpu.SemaphoreType.DMA] * (2 * _NBUF)
        ),
    )
    def agg(table, srcp2, dstp2, zeros, out, sidx, didx, *scr):
        rows = scr[:_NBUF]
        acc = scr[_NBUF]
        gsem = scr[_NBUF + 1:2 * _NBUF + 1]
        ssem = scr[2 * _NBUF + 1:]
        c = lax.axis_index("c")
        s = lax.axis_index("s")
        wid = c * _NS + s
        rbase = s * _RPT
        pltpu.sync_copy(zeros.at[pl.ds(rbase, _RPT)], acc.at[pl.ds(rbase, _RPT)])
        pltpu.sync_copy(srcp2.at[pl.ds(wid * _NCH, _NCH)], sidx)
        pltpu.sync_copy(dstp2.at[pl.ds(wid * _NCH, _NCH)], didx)
        plsc.subcore_barrier()

        def group(g, carry):
            jb = g * _NBUF
            for b in range(_NBUF):
                pltpu.async_copy(table.at[sidx.at[jb + b]], rows[b], gsem[b])
            for b in range(_NBUF):
                pltpu.make_async_copy(
                    table.at[sidx.at[jb + b]], rows[b], gsem[b]).wait()
                pltpu.async_copy(rows[b], acc.at[didx.at[jb + b]],
                                 ssem[b], add=True)
            for b in range(_NBUF):
                pltpu.make_async_copy(
                    rows[b], acc.at[didx.at[jb + b]], ssem[b]).wait()
            return carry

        lax.fori_loop(0, G, group, 0)
        plsc.subcore_barrier()
        pltpu.sync_copy(acc.at[pl.ds(rbase, _RPT)],
                        out.at[pl.ds(c * _NACC + rbase, _RPT)])

    return agg


_DEGK = 8


@functools.partial(
    pl.kernel,
    out_type=jax.ShapeDtypeStruct((_NC * 2 * _NACC, 16), jnp.float32),
    mesh=_sc_mesh(),
    compiler_params=pltpu.CompilerParams(use_tc_tiling_on_sc=False),
    scratch_types=[
        pltpu.VMEM((_NCH, _C), jnp.int32),
        pltpu.VMEM((_NCH, _C), jnp.int32),
        pltpu.VMEM((_C, 16), jnp.float32),
        pltpu.VMEM_SHARED((_NACC, 16), jnp.float32),
        pltpu.VMEM_SHARED((_NACC, 16), jnp.float32),
        pltpu.SemaphoreType.DMA,
        pltpu.SemaphoreType.DMA,
    ],
)
def _deg_kernel(srcp2, dstp2, ones, zeros, out, sidx, didx, ones_v,
                acca, accb, sem0, sem1):
    sems = (sem0, sem1)
    c = lax.axis_index("c")
    s = lax.axis_index("s")
    wid = c * _NS + s
    rbase = s * _RPT
    pltpu.sync_copy(ones, ones_v)
    pltpu.sync_copy(zeros.at[pl.ds(rbase, _RPT)], acca.at[pl.ds(rbase, _RPT)])
    pltpu.sync_copy(zeros.at[pl.ds(rbase, _RPT)], accb.at[pl.ds(rbase, _RPT)])
    pltpu.sync_copy(srcp2.at[pl.ds(wid * _NCH, _NCH)], sidx)
    pltpu.sync_copy(dstp2.at[pl.ds(wid * _NCH, _NCH)], didx)
    plsc.subcore_barrier()
    K = _DEGK // 2
    NG = _NCH // K

    def dgroup(g, carry):
        for k in range(K):
            pltpu.async_copy(ones_v, acca.at[sidx.at[g * K + k]],
                             sems[0], add=True)
            pltpu.async_copy(ones_v, accb.at[didx.at[g * K + k]],
                             sems[1], add=True)
        for k in range(K):
            pltpu.make_async_copy(
                ones_v, acca.at[sidx.at[g * K + k]], sems[0]).wait()
            pltpu.make_async_copy(
                ones_v, accb.at[didx.at[g * K + k]], sems[1]).wait()
        return carry

    lax.fori_loop(0, NG, dgroup, 0)
    plsc.subcore_barrier()
    pltpu.sync_copy(acca.at[pl.ds(rbase, _RPT)],
                    out.at[pl.ds(2 * c * _NACC + rbase, _RPT)])
    pltpu.sync_copy(accb.at[pl.ds(rbase, _RPT)],
                    out.at[pl.ds((2 * c + 1) * _NACC + rbase, _RPT)])


def _tc_norm_scale(xpad, degflat):
    def body(x_ref, sa_ref, sb_ref, da_ref, db_ref, xsa_ref, xsb_ref,
             ns_ref, nd_ref):
        ns = lax.rsqrt(jnp.maximum(sa_ref[:, 0:1] + sb_ref[:, 0:1], 1.0))
        nd = lax.rsqrt(jnp.maximum(da_ref[:, 0:1] + db_ref[:, 0:1], 1.0))
        xs = x_ref[...] * ns
        xsa_ref[...] = xs[:, :_D // 2]
        xsb_ref[...] = xs[:, _D // 2:]
        ns_ref[...] = ns
        nd_ref[...] = nd

    H = _D // 2
    return pl.pallas_call(
        body,
        grid=(_GRID,),
        in_specs=[
            pl.BlockSpec((_BR, _D), lambda i: (i, 0)),
            pl.BlockSpec((_BR, 16), lambda i: (i, 0)),
            pl.BlockSpec((_BR, 16), lambda i: (2 * _GRID + i, 0)),
            pl.BlockSpec((_BR, 16), lambda i: (_GRID + i, 0)),
            pl.BlockSpec((_BR, 16), lambda i: (3 * _GRID + i, 0)),
        ],
        out_specs=[
            pl.BlockSpec((_BR, H), lambda i: (i, 0)),
            pl.BlockSpec((_BR, H), lambda i: (i, 0)),
            pl.BlockSpec((_BR, 1), lambda i: (i, 0)),
            pl.BlockSpec((_BR, 1), lambda i: (i, 0)),
        ],
        out_shape=[
            jax.ShapeDtypeStruct((_NACC, H), jnp.float32),
            jax.ShapeDtypeStruct((_NACC, H), jnp.float32),
            jax.ShapeDtypeStruct((_NACC, 1), jnp.float32),
            jax.ShapeDtypeStruct((_NACC, 1), jnp.float32),
        ],
    )(xpad, degflat, degflat, degflat, degflat)


def _part_specs():
    H = _D // 2
    return [
        pl.BlockSpec((_BR, H), lambda i: (i, 0)),
        pl.BlockSpec((_BR, H), lambda i: (_GRID + i, 0)),
        pl.BlockSpec((_BR, H), lambda i: (i, 0)),
        pl.BlockSpec((_BR, H), lambda i: (_GRID + i, 0)),
    ]


def _tc_layer(pa, pb, nd, ns, W, b):
    def body(pa0, pa1, pb0, pb1, nd_ref, ns_ref, w_ref, b_ref, oa_ref, ob_ref):
        h = jnp.concatenate(
            [pa0[...] + pa1[...], pb0[...] + pb1[...]], axis=-1
        ) * nd_ref[...]
        z = jnp.dot(h, w_ref[...], preferred_element_type=jnp.float32) + b_ref[...]
        e = jnp.where(z > 0.0, z, jnp.exp(z) - 1.0) * ns_ref[...]
        oa_ref[...] = e[:, :_D // 2]
        ob_ref[...] = e[:, _D // 2:]

    H = _D // 2
    return pl.pallas_call(
        body,
        grid=(_GRID,),
        in_specs=_part_specs() + [
            pl.BlockSpec((_BR, 1), lambda i: (i, 0)),
            pl.BlockSpec((_BR, 1), lambda i: (i, 0)),
            pl.BlockSpec((_D, _D), lambda i: (0, 0)),
            pl.BlockSpec((1, _D), lambda i: (0, 0)),
        ],
        out_specs=[
            pl.BlockSpec((_BR, H), lambda i: (i, 0)),
            pl.BlockSpec((_BR, H), lambda i: (i, 0)),
        ],
        out_shape=[
            jax.ShapeDtypeStruct((_NACC, H), jnp.float32),
            jax.ShapeDtypeStruct((_NACC, H), jnp.float32),
        ],
    )(pa, pa, pb, pb, nd, ns, W, b)


def _tc_layer_premat(pa, pb, nd, ns, W, b, W2p):
    def body(pa0, pa1, pb0, pb1, nd_ref, ns_ref, w_ref, b_ref, w2_ref, o_ref):
        h = jnp.concatenate(
            [pa0[...] + pa1[...], pb0[...] + pb1[...]], axis=-1
        ) * nd_ref[...]
        z = jnp.dot(h, w_ref[...], preferred_element_type=jnp.float32) + b_ref[...]
        h2 = jnp.where(z > 0.0, z, jnp.exp(z) - 1.0) * ns_ref[...]
        o_ref[...] = jnp.dot(h2, w2_ref[...], preferred_element_type=jnp.float32)

    return pl.pallas_call(
        body,
        grid=(_GRID,),
        in_specs=_part_specs() + [
            pl.BlockSpec((_BR, 1), lambda i: (i, 0)),
            pl.BlockSpec((_BR, 1), lambda i: (i, 0)),
            pl.BlockSpec((_D, _D), lambda i: (0, 0)),
            pl.BlockSpec((1, _D), lambda i: (0, 0)),
            pl.BlockSpec((_D, _DC), lambda i: (0, 0)),
        ],
        out_specs=pl.BlockSpec((_BR, _DC), lambda i: (i, 0)),
        out_shape=jax.ShapeDtypeStruct((_NACC, _DC), jnp.float32),
    )(pa, pa, pb, pb, nd, ns, W, b, W2p)


def _tc_final(p2, nd, b2p):
    def body(p0, p1, nd_ref, b_ref, o_ref):
        z = (p0[...] + p1[...]) * nd_ref[...] + b_ref[...]
        o_ref[...] = z[:, :_NCLS]

    return pl.pallas_call(
        body,
        grid=(_GRID,),
        in_specs=[
            pl.BlockSpec((_BR, _DC), lambda i: (i, 0)),
            pl.BlockSpec((_BR, _DC), lambda i: (_GRID + i, 0)),
            pl.BlockSpec((_BR, 1), lambda i: (i, 0)),
            pl.BlockSpec((1, _DC), lambda i: (0, 0)),
        ],
        out_specs=pl.BlockSpec((_BR, _NCLS), lambda i: (i, 0)),
        out_shape=jax.ShapeDtypeStruct((_N, _NCLS), jnp.float32),
    )(p2, p2, nd, b2p)


def kernel(x, edge_index, W0, b0, W1, b1, W2, b2):
    src = edge_index[0]
    dst = edge_index[1]
    padi = _N + (jnp.arange(_EPAD - _E, dtype=jnp.int32) % (_NACC - _N))
    srcp2 = jnp.concatenate([src, padi]).reshape(_NW * _NCH, _C)
    dstp2 = jnp.concatenate([dst, padi]).reshape(_NW * _NCH, _C)
    ones16 = jnp.ones((_C, 16), jnp.float32)
    zdeg = jnp.zeros((_NACC, 16), jnp.float32)
    z64 = jnp.zeros((_NACC, _D // 2), jnp.float32)
    z48 = jnp.zeros((_NACC, _DC), jnp.float32)

    degflat = _deg_kernel(srcp2, dstp2, ones16, zdeg)
    xpad = jnp.pad(x, ((0, _NACC - _N), (0, 0)))
    xsa, xsb, ns, nd = _tc_norm_scale(xpad, degflat)

    agg64 = _make_agg(_D // 2)
    p0a = agg64(xsa, srcp2, dstp2, z64)
    p0b = agg64(xsb, srcp2, dstp2, z64)
    h1a, h1b = _tc_layer(p0a, p0b, nd, ns, W0, b0.reshape(1, _D))
    p1a = agg64(h1a, srcp2, dstp2, z64)
    p1b = agg64(h1b, srcp2, dstp2, z64)
    W2p = jnp.pad(W2, ((0, 0), (0, _DC - _NCLS)))
    z2 = _tc_layer_premat(p1a, p1b, nd, ns, W1, b1.reshape(1, _D), W2p)
    p2 = _make_agg(_DC)(z2, srcp2, dstp2, z48)
    b2p = jnp.pad(b2, (0, _DC - _NCLS)).reshape(1, _DC)
    return _tc_final(p2, nd, b2p)


# speedup vs baseline: 10.2266x; 1.0794x over previous
import functools

import jax
import jax.numpy as jnp
from jax import lax
from jax.experimental import pallas as pl
from jax.experimental.pallas import tpu as pltpu
from jax.experimental.pallas import tpu_sc as plsc

_N = 10000
_NACC = 10240
_D = 128
_NCLS = 40
_DC = 48
_E = 320000
_EPAD = 327680
_NC = 2
_NS = 16
_NW = _NC * _NS
_EPT = _EPAD // _NW
_C = 128
_NCH = _EPT // _C
_RPT = _NACC // _NS
_L2 = 2 * _NACC
_EPT2 = 2 * _EPT
_NCH2 = _EPT2 // _C
_RPT2 = _L2 // _NS
_BR = 512
_GRID = _NACC // _BR


def _sc_mesh():
    return plsc.VectorSubcoreMesh(core_axis_name="c", subcore_axis_name="s")


_NBUF = 4
_NCHF = _EPAD // _NS // _C


def _agg_feature_split():
    D = _D // 2
    G = _NCHF // _NBUF

    @functools.partial(
        pl.kernel,
        out_type=jax.ShapeDtypeStruct((_NC * _NACC, D), jnp.float32),
        mesh=_sc_mesh(),
        compiler_params=pltpu.CompilerParams(use_tc_tiling_on_sc=False),
        scratch_types=(
            [pltpu.VMEM((_NCHF, _C), jnp.int32)] * 2
            + [pltpu.VMEM((_C, D), jnp.float32)] * _NBUF
            + [pltpu.VMEM_SHARED((_NACC, D), jnp.float32)]
            + [pltpu.SemaphoreType.DMA] * (2 * _NBUF)
        ),
    )
    def agg(table_lo, table_hi, srcp2, dstp2, zeros, out, sidx, didx, *scr):
        rows = scr[:_NBUF]
        acc = scr[_NBUF]
        gsem = scr[_NBUF + 1:2 * _NBUF + 1]
        ssem = scr[2 * _NBUF + 1:]
        c = lax.axis_index("c")
        s = lax.axis_index("s")
        rbase = s * _RPT
        pltpu.sync_copy(zeros.at[pl.ds(rbase, _RPT)], acc.at[pl.ds(rbase, _RPT)])
        pltpu.sync_copy(srcp2.at[pl.ds(s * _NCHF, _NCHF)], sidx)
        pltpu.sync_copy(dstp2.at[pl.ds(s * _NCHF, _NCHF)], didx)
        plsc.subcore_barrier()

        def run(table):
            def group(g, carry):
                jb = g * _NBUF
                for b in range(_NBUF):
                    pltpu.async_copy(table.at[sidx.at[jb + b]], rows[b], gsem[b])
                for b in range(_NBUF):
                    pltpu.make_async_copy(
                        table.at[sidx.at[jb + b]], rows[b], gsem[b]).wait()
                    pltpu.async_copy(rows[b], acc.at[didx.at[jb + b]],
                                     ssem[b], add=True)
                for b in range(_NBUF):
                    pltpu.make_async_copy(
                        rows[b], acc.at[didx.at[jb + b]], ssem[b]).wait()
                return carry

            lax.fori_loop(0, G, group, 0)

        pl.when(c == 0)(lambda: run(table_lo))
        pl.when(c == 1)(lambda: run(table_hi))
        plsc.subcore_barrier()
        pltpu.sync_copy(acc.at[pl.ds(rbase, _RPT)],
                        out.at[pl.ds(c * _NACC + rbase, _RPT)])

    return agg


@functools.lru_cache(maxsize=None)
def _make_agg(D):
    G = _NCH // _NBUF

    @functools.partial(
        pl.kernel,
        out_type=jax.ShapeDtypeStruct((_NC * _NACC, D), jnp.float32),
        mesh=_sc_mesh(),
        compiler_params=pltpu.CompilerParams(use_tc_tiling_on_sc=(D % 128 == 0)),
        scratch_types=(
            [pltpu.VMEM((_NCH, _C), jnp.int32)] * 2
            + [pltpu.VMEM((_C, D), jnp.float32)] * _NBUF
            + [pltpu.VMEM_SHARED((_NACC, D), jnp.float32)]
            + [pltpu.SemaphoreType.DMA] * (2 * _NBUF)
        ),
    )
    def agg(table, srcp2, dstp2, zeros, out, sidx, didx, *scr):
        rows = scr[:_NBUF]
        acc = scr[_NBUF]
        gsem = scr[_NBUF + 1:2 * _NBUF + 1]
        ssem = scr[2 * _NBUF + 1:]
        c = lax.axis_index("c")
        s = lax.axis_index("s")
        wid = c * _NS + s
        rbase = s * _RPT
        pltpu.sync_copy(zeros.at[pl.ds(rbase, _RPT)], acc.at[pl.ds(rbase, _RPT)])
        pltpu.sync_copy(srcp2.at[pl.ds(wid * _NCH, _NCH)], sidx)
        pltpu.sync_copy(dstp2.at[pl.ds(wid * _NCH, _NCH)], didx)
        plsc.subcore_barrier()

        def group(g, carry):
            jb = g * _NBUF
            for b in range(_NBUF):
                pltpu.async_copy(table.at[sidx.at[jb + b]], rows[b], gsem[b])
            for b in range(_NBUF):
                pltpu.make_async_copy(
                    table.at[sidx.at[jb + b]], rows[b], gsem[b]).wait()
                pltpu.async_copy(rows[b], acc.at[didx.at[jb + b]],
                                 ssem[b], add=True)
            for b in range(_NBUF):
                pltpu.make_async_copy(
                    rows[b], acc.at[didx.at[jb + b]], ssem[b]).wait()
            return carry

        lax.fori_loop(0, G, group, 0)
        plsc.subcore_barrier()
        pltpu.sync_copy(acc.at[pl.ds(rbase, _RPT)],
                        out.at[pl.ds(c * _NACC + rbase, _RPT)])

    return agg


_DEGK = 8


@functools.partial(
    pl.kernel,
    out_type=jax.ShapeDtypeStruct((_NC * _NACC, 16), jnp.float32),
    mesh=_sc_mesh(),
    compiler_params=pltpu.CompilerParams(use_tc_tiling_on_sc=False),
    scratch_types=[
        pltpu.VMEM((_NCHF, _C), jnp.int32),
        pltpu.VMEM((_C, 16), jnp.float32),
        pltpu.VMEM_SHARED((_NACC, 16), jnp.float32),
        pltpu.SemaphoreType.DMA,
        pltpu.SemaphoreType.DMA,
    ],
)
def _deg_kernel(srcp2, dstp2, ones, zeros, out, idxbuf, ones_v,
                acc, sem0, sem1):
    sems = (sem0, sem1)
    c = lax.axis_index("c")
    s = lax.axis_index("s")
    rbase = s * _RPT
    pltpu.sync_copy(ones, ones_v)
    pltpu.sync_copy(zeros.at[pl.ds(rbase, _RPT)], acc.at[pl.ds(rbase, _RPT)])

    def load_idx(idx2):
        pltpu.sync_copy(idx2.at[pl.ds(s * _NCHF, _NCHF)], idxbuf)

    pl.when(c == 0)(lambda: load_idx(srcp2))
    pl.when(c == 1)(lambda: load_idx(dstp2))
    plsc.subcore_barrier()
    NG = _NCHF // _DEGK

    def dgroup(g, carry):
        for k in range(_DEGK):
            pltpu.async_copy(ones_v, acc.at[idxbuf.at[g * _DEGK + k]],
                             sems[k % 2], add=True)
        for k in range(_DEGK):
            pltpu.make_async_copy(
                ones_v, acc.at[idxbuf.at[g * _DEGK + k]], sems[k % 2]).wait()
        return carry

    lax.fori_loop(0, NG, dgroup, 0)
    plsc.subcore_barrier()
    pltpu.sync_copy(acc.at[pl.ds(rbase, _RPT)],
                    out.at[pl.ds(c * _NACC + rbase, _RPT)])


def _tc_norm_scale(xpad, degflat):
    def body(x_ref, ds_ref, dd_ref, xsa_ref, xsb_ref, ns_ref, nd_ref):
        ns = lax.rsqrt(jnp.maximum(ds_ref[:, 0:1], 1.0))
        nd = lax.rsqrt(jnp.maximum(dd_ref[:, 0:1], 1.0))
        xs = x_ref[...] * ns
        xsa_ref[...] = xs[:, :_D // 2]
        xsb_ref[...] = xs[:, _D // 2:]
        ns_ref[...] = ns
        nd_ref[...] = nd

    H = _D // 2
    return pl.pallas_call(
        body,
        grid=(_GRID,),
        in_specs=[
            pl.BlockSpec((_BR, _D), lambda i: (i, 0)),
            pl.BlockSpec((_BR, 16), lambda i: (i, 0)),
            pl.BlockSpec((_BR, 16), lambda i: (_GRID + i, 0)),
        ],
        out_specs=[
            pl.BlockSpec((_BR, H), lambda i: (i, 0)),
            pl.BlockSpec((_BR, H), lambda i: (i, 0)),
            pl.BlockSpec((_BR, 1), lambda i: (i, 0)),
            pl.BlockSpec((_BR, 1), lambda i: (i, 0)),
        ],
        out_shape=[
            jax.ShapeDtypeStruct((_NACC, H), jnp.float32),
            jax.ShapeDtypeStruct((_NACC, H), jnp.float32),
            jax.ShapeDtypeStruct((_NACC, 1), jnp.float32),
            jax.ShapeDtypeStruct((_NACC, 1), jnp.float32),
        ],
    )(xpad, degflat, degflat)


def _half_specs():
    H = _D // 2
    return [
        pl.BlockSpec((_BR, H), lambda i: (i, 0)),
        pl.BlockSpec((_BR, H), lambda i: (_GRID + i, 0)),
    ]


def _tc_layer(p, nd, ns, W, b):
    def body(plo, phi, nd_ref, ns_ref, w_ref, b_ref, oa_ref, ob_ref):
        h = jnp.concatenate([plo[...], phi[...]], axis=-1) * nd_ref[...]
        z = jnp.dot(h, w_ref[...], preferred_element_type=jnp.float32) + b_ref[...]
        e = jnp.where(z > 0.0, z, jnp.exp(z) - 1.0) * ns_ref[...]
        oa_ref[...] = e[:, :_D // 2]
        ob_ref[...] = e[:, _D // 2:]

    H = _D // 2
    return pl.pallas_call(
        body,
        grid=(_GRID,),
        in_specs=_half_specs() + [
            pl.BlockSpec((_BR, 1), lambda i: (i, 0)),
            pl.BlockSpec((_BR, 1), lambda i: (i, 0)),
            pl.BlockSpec((_D, _D), lambda i: (0, 0)),
            pl.BlockSpec((1, _D), lambda i: (0, 0)),
        ],
        out_specs=[
            pl.BlockSpec((_BR, H), lambda i: (i, 0)),
            pl.BlockSpec((_BR, H), lambda i: (i, 0)),
        ],
        out_shape=[
            jax.ShapeDtypeStruct((_NACC, H), jnp.float32),
            jax.ShapeDtypeStruct((_NACC, H), jnp.float32),
        ],
    )(p, p, nd, ns, W, b)


def _tc_layer_premat(p, nd, ns, W, b, W2p):
    def body(plo, phi, nd_ref, ns_ref, w_ref, b_ref, w2_ref, o_ref):
        h = jnp.concatenate([plo[...], phi[...]], axis=-1) * nd_ref[...]
        z = jnp.dot(h, w_ref[...], preferred_element_type=jnp.float32) + b_ref[...]
        h2 = jnp.where(z > 0.0, z, jnp.exp(z) - 1.0) * ns_ref[...]
        o_ref[...] = jnp.dot(h2, w2_ref[...], preferred_element_type=jnp.float32)

    return pl.pallas_call(
        body,
        grid=(_GRID,),
        in_specs=_half_specs() + [
            pl.BlockSpec((_BR, 1), lambda i: (i, 0)),
            pl.BlockSpec((_BR, 1), lambda i: (i, 0)),
            pl.BlockSpec((_D, _D), lambda i: (0, 0)),
            pl.BlockSpec((1, _D), lambda i: (0, 0)),
            pl.BlockSpec((_D, _DC), lambda i: (0, 0)),
        ],
        out_specs=pl.BlockSpec((_BR, _DC), lambda i: (i, 0)),
        out_shape=jax.ShapeDtypeStruct((_NACC, _DC), jnp.float32),
    )(p, p, nd, ns, W, b, W2p)


def _tc_final(p2, nd, b2p):
    def body(p0, p1, nd_ref, b_ref, o_ref):
        z = (p0[...] + p1[...]) * nd_ref[...] + b_ref[...]
        o_ref[...] = z[:, :_NCLS]

    return pl.pallas_call(
        body,
        grid=(_GRID,),
        in_specs=[
            pl.BlockSpec((_BR, _DC), lambda i: (i, 0)),
            pl.BlockSpec((_BR, _DC), lambda i: (_GRID + i, 0)),
            pl.BlockSpec((_BR, 1), lambda i: (i, 0)),
            pl.BlockSpec((1, _DC), lambda i: (0, 0)),
        ],
        out_specs=pl.BlockSpec((_BR, _NCLS), lambda i: (i, 0)),
        out_shape=jax.ShapeDtypeStruct((_N, _NCLS), jnp.float32),
    )(p2, p2, nd, b2p)


def kernel(x, edge_index, W0, b0, W1, b1, W2, b2):
    src = edge_index[0]
    dst = edge_index[1]
    padi = _N + (jnp.arange(_EPAD - _E, dtype=jnp.int32) % (_NACC - _N))
    srcp2 = jnp.concatenate([src, padi]).reshape(_NW * _NCH, _C)
    dstp2 = jnp.concatenate([dst, padi]).reshape(_NW * _NCH, _C)
    ones16 = jnp.ones((_C, 16), jnp.float32)
    zdeg = jnp.zeros((_NACC, 16), jnp.float32)
    z64 = jnp.zeros((_NACC, _D // 2), jnp.float32)
    z48 = jnp.zeros((_NACC, _DC), jnp.float32)

    degflat = _deg_kernel(srcp2, dstp2, ones16, zdeg)
    xpad = jnp.pad(x, ((0, _NACC - _N), (0, 0)))
    xsa, xsb, ns, nd = _tc_norm_scale(xpad, degflat)

    aggf = _agg_feature_split()
    p0 = aggf(xsa, xsb, srcp2, dstp2, z64)
    h1a, h1b = _tc_layer(p0, nd, ns, W0, b0.reshape(1, _D))
    p1 = aggf(h1a, h1b, srcp2, dstp2, z64)
    W2p = jnp.pad(W2, ((0, 0), (0, _DC - _NCLS)))
    z2 = _tc_layer_premat(p1, nd, ns, W1, b1.reshape(1, _D), W2p)
    p2 = _make_agg(_DC)(z2, srcp2, dstp2, z48)
    b2p = jnp.pad(b2, (0, _DC - _NCLS)).reshape(1, _DC)
    return _tc_final(p2, nd, b2p)


# speedup vs baseline: 11.5957x; 1.1339x over previous
import functools

import jax
import jax.numpy as jnp
from jax import lax
from jax.experimental import pallas as pl
from jax.experimental.pallas import tpu as pltpu
from jax.experimental.pallas import tpu_sc as plsc

_N = 10000
_NACC = 10240
_D = 128
_NCLS = 40
_DC = 48
_E = 320000
_EPAD = 327680
_NC = 2
_NS = 16
_NW = _NC * _NS
_EPT = _EPAD // _NW
_C = 128
_NCH = _EPT // _C
_RPT = _NACC // _NS
_L2 = 2 * _NACC
_EPT2 = 2 * _EPT
_NCH2 = _EPT2 // _C
_RPT2 = _L2 // _NS
_BR = 512
_GRID = _NACC // _BR


def _sc_mesh():
    return plsc.VectorSubcoreMesh(core_axis_name="c", subcore_axis_name="s")


_NBUF = 4
_NCHF = _EPAD // _NS // _C


def _agg_feature_split():
    D = _D // 2
    G = _NCHF // (2 * _NBUF)

    @functools.partial(
        pl.kernel,
        out_type=jax.ShapeDtypeStruct((_NACC, _D), jnp.float32),
        mesh=_sc_mesh(),
        compiler_params=pltpu.CompilerParams(use_tc_tiling_on_sc=False),
        scratch_types=(
            [pltpu.VMEM((_NCHF, _C), jnp.int32)] * 2
            + [pltpu.VMEM((_C, D), jnp.float32)] * _NBUF
            + [pltpu.VMEM_SHARED((_NACC, D), jnp.float32)]
            + [pltpu.SemaphoreType.DMA] * (2 * _NBUF)
        ),
    )
    def agg(table_lo, table_hi, srcp2, dstp2, zeros, out, sidx, didx, *scr):
        rows = scr[:_NBUF]
        acc = scr[_NBUF]
        gsem = scr[_NBUF + 1:2 * _NBUF + 1]
        ssem = scr[2 * _NBUF + 1:]
        c = lax.axis_index("c")
        s = lax.axis_index("s")
        rbase = s * _RPT
        pltpu.sync_copy(zeros.at[pl.ds(rbase, _RPT)], acc.at[pl.ds(rbase, _RPT)])
        pltpu.sync_copy(srcp2.at[pl.ds(s * _NCHF, _NCHF)], sidx)
        pltpu.sync_copy(dstp2.at[pl.ds(s * _NCHF, _NCHF)], didx)
        plsc.subcore_barrier()

        def run(table):
            def gather(j, b):
                pltpu.async_copy(table.at[sidx.at[j]], rows[b], gsem[b])

            def gwait_scat(j, b):
                pltpu.make_async_copy(
                    table.at[sidx.at[j]], rows[b], gsem[b]).wait()
                pltpu.async_copy(rows[b], acc.at[didx.at[j]], ssem[b], add=True)

            def swait(j, b):
                pltpu.make_async_copy(
                    rows[b], acc.at[didx.at[j]], ssem[b]).wait()

            def group(g, carry):
                ja = g * 2 * _NBUF
                jb = ja + _NBUF
                for b in range(_NBUF):
                    gather(ja + b, b)
                for b in range(_NBUF):
                    gwait_scat(ja + b, b)
                for b in range(_NBUF):
                    swait(ja + b, b)
                    gather(jb + b, b)
                for b in range(_NBUF):
                    gwait_scat(jb + b, b)
                for b in range(_NBUF):
                    swait(jb + b, b)
                return carry

            lax.fori_loop(0, G, group, 0)

        pl.when(c == 0)(lambda: run(table_lo))
        pl.when(c == 1)(lambda: run(table_hi))
        plsc.subcore_barrier()
        pltpu.sync_copy(acc.at[pl.ds(rbase, _RPT)],
                        out.at[pl.ds(rbase, _RPT), pl.ds(c * D, D)])

    return agg


@functools.lru_cache(maxsize=None)
def _make_agg(D):
    G = _NCH // _NBUF

    @functools.partial(
        pl.kernel,
        out_type=jax.ShapeDtypeStruct((_NC * _NACC, D), jnp.float32),
        mesh=_sc_mesh(),
        compiler_params=pltpu.CompilerParams(use_tc_tiling_on_sc=(D % 128 == 0)),
        scratch_types=(
            [pltpu.VMEM((_NCH, _C), jnp.int32)] * 2
            + [pltpu.VMEM((_C, D), jnp.float32)] * _NBUF
            + [pltpu.VMEM_SHARED((_NACC, D), jnp.float32)]
            + [pltpu.SemaphoreType.DMA] * (2 * _NBUF)
        ),
    )
    def agg(table, srcp2, dstp2, zeros, out, sidx, didx, *scr):
        rows = scr[:_NBUF]
        acc = scr[_NBUF]
        gsem = scr[_NBUF + 1:2 * _NBUF + 1]
        ssem = scr[2 * _NBUF + 1:]
        c = lax.axis_index("c")
        s = lax.axis_index("s")
        wid = c * _NS + s
        rbase = s * _RPT
        pltpu.sync_copy(zeros.at[pl.ds(rbase, _RPT)], acc.at[pl.ds(rbase, _RPT)])
        pltpu.sync_copy(srcp2.at[pl.ds(wid * _NCH, _NCH)], sidx)
        pltpu.sync_copy(dstp2.at[pl.ds(wid * _NCH, _NCH)], didx)
        plsc.subcore_barrier()

        def gather(j, b):
            pltpu.async_copy(table.at[sidx.at[j]], rows[b], gsem[b])

        def gwait_scat(j, b):
            pltpu.make_async_copy(table.at[sidx.at[j]], rows[b], gsem[b]).wait()
            pltpu.async_copy(rows[b], acc.at[didx.at[j]], ssem[b], add=True)

        def swait(j, b):
            pltpu.make_async_copy(rows[b], acc.at[didx.at[j]], ssem[b]).wait()

        def group(g, carry):
            ja = g * 2 * _NBUF
            jb = ja + _NBUF
            for b in range(_NBUF):
                gather(ja + b, b)
            for b in range(_NBUF):
                gwait_scat(ja + b, b)
            for b in range(_NBUF):
                swait(ja + b, b)
                gather(jb + b, b)
            for b in range(_NBUF):
                gwait_scat(jb + b, b)
            for b in range(_NBUF):
                swait(jb + b, b)
            return carry

        lax.fori_loop(0, _NCH // (2 * _NBUF), group, 0)
        plsc.subcore_barrier()
        pltpu.sync_copy(acc.at[pl.ds(rbase, _RPT)],
                        out.at[pl.ds(c * _NACC + rbase, _RPT)])

    return agg


_DEGK = 8


@functools.partial(
    pl.kernel,
    out_type=jax.ShapeDtypeStruct((_NC * _NACC, 16), jnp.float32),
    mesh=_sc_mesh(),
    compiler_params=pltpu.CompilerParams(use_tc_tiling_on_sc=False),
    scratch_types=[
        pltpu.VMEM((_NCHF, _C), jnp.int32),
        pltpu.VMEM((_C, 16), jnp.float32),
        pltpu.VMEM_SHARED((_NACC, 16), jnp.float32),
        pltpu.SemaphoreType.DMA,
        pltpu.SemaphoreType.DMA,
    ],
)
def _deg_kernel(srcp2, dstp2, ones, zeros, out, idxbuf, ones_v,
                acc, sem0, sem1):
    sems = (sem0, sem1)
    c = lax.axis_index("c")
    s = lax.axis_index("s")
    rbase = s * _RPT
    pltpu.sync_copy(ones, ones_v)
    pltpu.sync_copy(zeros.at[pl.ds(rbase, _RPT)], acc.at[pl.ds(rbase, _RPT)])

    def load_idx(idx2):
        pltpu.sync_copy(idx2.at[pl.ds(s * _NCHF, _NCHF)], idxbuf)

    pl.when(c == 0)(lambda: load_idx(srcp2))
    pl.when(c == 1)(lambda: load_idx(dstp2))
    plsc.subcore_barrier()
    NG = _NCHF // _DEGK

    def dgroup(g, carry):
        for k in range(_DEGK):
            pltpu.async_copy(ones_v, acc.at[idxbuf.at[g * _DEGK + k]],
                             sems[k % 2], add=True)
        for k in range(_DEGK):
            pltpu.make_async_copy(
                ones_v, acc.at[idxbuf.at[g * _DEGK + k]], sems[k % 2]).wait()
        return carry

    lax.fori_loop(0, NG, dgroup, 0)
    plsc.subcore_barrier()
    pltpu.sync_copy(acc.at[pl.ds(rbase, _RPT)],
                    out.at[pl.ds(c * _NACC + rbase, _RPT)])


def _tc_norm_scale(xpad, degflat):
    def body(x_ref, ds_ref, dd_ref, xsa_ref, xsb_ref, ns_ref, nd_ref):
        ns = lax.rsqrt(jnp.maximum(ds_ref[:, 0:1], 1.0))
        nd = lax.rsqrt(jnp.maximum(dd_ref[:, 0:1], 1.0))
        xs = x_ref[...] * ns
        xsa_ref[...] = xs[:, :_D // 2]
        xsb_ref[...] = xs[:, _D // 2:]
        ns_ref[...] = ns
        nd_ref[...] = nd

    H = _D // 2
    return pl.pallas_call(
        body,
        grid=(_GRID,),
        in_specs=[
            pl.BlockSpec((_BR, _D), lambda i: (i, 0)),
            pl.BlockSpec((_BR, 16), lambda i: (i, 0)),
            pl.BlockSpec((_BR, 16), lambda i: (_GRID + i, 0)),
        ],
        out_specs=[
            pl.BlockSpec((_BR, H), lambda i: (i, 0)),
            pl.BlockSpec((_BR, H), lambda i: (i, 0)),
            pl.BlockSpec((_BR, 1), lambda i: (i, 0)),
            pl.BlockSpec((_BR, 1), lambda i: (i, 0)),
        ],
        out_shape=[
            jax.ShapeDtypeStruct((_NACC, H), jnp.float32),
            jax.ShapeDtypeStruct((_NACC, H), jnp.float32),
            jax.ShapeDtypeStruct((_NACC, 1), jnp.float32),
            jax.ShapeDtypeStruct((_NACC, 1), jnp.float32),
        ],
    )(xpad, degflat, degflat)


def _tc_layer(p, nd, ns, W, b):
    def body(p_ref, nd_ref, ns_ref, w_ref, b_ref, oa_ref, ob_ref):
        h = p_ref[...] * nd_ref[...]
        z = jnp.dot(h, w_ref[...], preferred_element_type=jnp.float32) + b_ref[...]
        e = jnp.where(z > 0.0, z, jnp.exp(z) - 1.0) * ns_ref[...]
        oa_ref[...] = e[:, :_D // 2]
        ob_ref[...] = e[:, _D // 2:]

    H = _D // 2
    return pl.pallas_call(
        body,
        grid=(_GRID,),
        in_specs=[
            pl.BlockSpec((_BR, _D), lambda i: (i, 0)),
            pl.BlockSpec((_BR, 1), lambda i: (i, 0)),
            pl.BlockSpec((_BR, 1), lambda i: (i, 0)),
            pl.BlockSpec((_D, _D), lambda i: (0, 0)),
            pl.BlockSpec((1, _D), lambda i: (0, 0)),
        ],
        out_specs=[
            pl.BlockSpec((_BR, H), lambda i: (i, 0)),
            pl.BlockSpec((_BR, H), lambda i: (i, 0)),
        ],
        out_shape=[
            jax.ShapeDtypeStruct((_NACC, H), jnp.float32),
            jax.ShapeDtypeStruct((_NACC, H), jnp.float32),
        ],
    )(p, nd, ns, W, b)


def _tc_layer_premat(p, nd, ns, W, b, W2p):
    def body(p_ref, nd_ref, ns_ref, w_ref, b_ref, w2_ref, o_ref):
        h = p_ref[...] * nd_ref[...]
        z = jnp.dot(h, w_ref[...], preferred_element_type=jnp.float32) + b_ref[...]
        h2 = jnp.where(z > 0.0, z, jnp.exp(z) - 1.0) * ns_ref[...]
        o_ref[...] = jnp.dot(h2, w2_ref[...], preferred_element_type=jnp.float32)

    return pl.pallas_call(
        body,
        grid=(_GRID,),
        in_specs=[
            pl.BlockSpec((_BR, _D), lambda i: (i, 0)),
            pl.BlockSpec((_BR, 1), lambda i: (i, 0)),
            pl.BlockSpec((_BR, 1), lambda i: (i, 0)),
            pl.BlockSpec((_D, _D), lambda i: (0, 0)),
            pl.BlockSpec((1, _D), lambda i: (0, 0)),
            pl.BlockSpec((_D, _DC), lambda i: (0, 0)),
        ],
        out_specs=pl.BlockSpec((_BR, _DC), lambda i: (i, 0)),
        out_shape=jax.ShapeDtypeStruct((_NACC, _DC), jnp.float32),
    )(p, nd, ns, W, b, W2p)


def _tc_final(p2, nd, b2p):
    def body(p0, p1, nd_ref, b_ref, o_ref):
        z = (p0[...] + p1[...]) * nd_ref[...] + b_ref[...]
        o_ref[...] = z[:, :_NCLS]

    return pl.pallas_call(
        body,
        grid=(_GRID,),
        in_specs=[
            pl.BlockSpec((_BR, _DC), lambda i: (i, 0)),
            pl.BlockSpec((_BR, _DC), lambda i: (_GRID + i, 0)),
            pl.BlockSpec((_BR, 1), lambda i: (i, 0)),
            pl.BlockSpec((1, _DC), lambda i: (0, 0)),
        ],
        out_specs=pl.BlockSpec((_BR, _NCLS), lambda i: (i, 0)),
        out_shape=jax.ShapeDtypeStruct((_N, _NCLS), jnp.float32),
    )(p2, p2, nd, b2p)


def kernel(x, edge_index, W0, b0, W1, b1, W2, b2):
    src = edge_index[0]
    dst = edge_index[1]
    padi = _N + (jnp.arange(_EPAD - _E, dtype=jnp.int32) % (_NACC - _N))
    srcp2 = jnp.concatenate([src, padi]).reshape(_NW * _NCH, _C)
    dstp2 = jnp.concatenate([dst, padi]).reshape(_NW * _NCH, _C)
    ones16 = jnp.ones((_C, 16), jnp.float32)
    zdeg = jnp.zeros((_NACC, 16), jnp.float32)
    z64 = jnp.zeros((_NACC, _D // 2), jnp.float32)
    z48 = jnp.zeros((_NACC, _DC), jnp.float32)

    degflat = _deg_kernel(srcp2, dstp2, ones16, zdeg)
    xpad = jnp.pad(x, ((0, _NACC - _N), (0, 0)))
    xsa, xsb, ns, nd = _tc_norm_scale(xpad, degflat)

    aggf = _agg_feature_split()
    p0 = aggf(xsa, xsb, srcp2, dstp2, z64)
    h1a, h1b = _tc_layer(p0, nd, ns, W0, b0.reshape(1, _D))
    p1 = aggf(h1a, h1b, srcp2, dstp2, z64)
    W2p = jnp.pad(W2, ((0, 0), (0, _DC - _NCLS)))
    z2 = _tc_layer_premat(p1, nd, ns, W1, b1.reshape(1, _D), W2p)
    p2 = _make_agg(_DC)(z2, srcp2, dstp2, z48)
    b2p = jnp.pad(b2, (0, _DC - _NCLS)).reshape(1, _DC)
    return _tc_final(p2, nd, b2p)


# speedup vs baseline: 12.4453x; 1.0733x over previous
import functools

import jax
import jax.numpy as jnp
from jax import lax
from jax.experimental import pallas as pl
from jax.experimental.pallas import tpu as pltpu
from jax.experimental.pallas import tpu_sc as plsc

_N = 10000
_NACC = 10240
_D = 128
_NCLS = 40
_DC = 48
_E = 320000
_EPAD = 327680
_NC = 2
_NS = 16
_NW = _NC * _NS
_EPT = _EPAD // _NW
_C = 128
_NCH = _EPT // _C
_RPT = _NACC // _NS
_L2 = 2 * _NACC
_EPT2 = 2 * _EPT
_NCH2 = _EPT2 // _C
_RPT2 = _L2 // _NS
_BR = 512
_GRID = _NACC // _BR


def _sc_mesh():
    return plsc.VectorSubcoreMesh(core_axis_name="c", subcore_axis_name="s")


_NBUF = 4
_NCHF = _EPAD // _NS // _C


def _agg_feature_split():
    D = _D // 2
    G = _NCHF // (2 * _NBUF)

    @functools.partial(
        pl.kernel,
        out_type=jax.ShapeDtypeStruct((_NACC, _D), jnp.float32),
        mesh=_sc_mesh(),
        compiler_params=pltpu.CompilerParams(use_tc_tiling_on_sc=False),
        scratch_types=(
            [pltpu.VMEM((_NCHF, _C), jnp.int32)] * 2
            + [pltpu.VMEM((_C, D), jnp.float32)] * _NBUF
            + [pltpu.VMEM_SHARED((_NACC, D), jnp.float32)]
            + [pltpu.SemaphoreType.DMA] * (2 * _NBUF)
        ),
    )
    def agg(table, s2lo, s2hi, dstp2, zeros, out, sidx, didx, *scr):
        rows = scr[:_NBUF]
        acc = scr[_NBUF]
        gsem = scr[_NBUF + 1:2 * _NBUF + 1]
        ssem = scr[2 * _NBUF + 1:]
        c = lax.axis_index("c")
        s = lax.axis_index("s")
        rbase = s * _RPT
        pltpu.sync_copy(zeros.at[pl.ds(rbase, _RPT)], acc.at[pl.ds(rbase, _RPT)])
        pl.when(c == 0)(
            lambda: pltpu.sync_copy(s2lo.at[pl.ds(s * _NCHF, _NCHF)], sidx))
        pl.when(c == 1)(
            lambda: pltpu.sync_copy(s2hi.at[pl.ds(s * _NCHF, _NCHF)], sidx))
        pltpu.sync_copy(dstp2.at[pl.ds(s * _NCHF, _NCHF)], didx)
        plsc.subcore_barrier()

        def gather(j, b):
            pltpu.async_copy(table.at[sidx.at[j]], rows[b], gsem[b])

        def gwait_scat(j, b):
            pltpu.make_async_copy(table.at[sidx.at[j]], rows[b], gsem[b]).wait()
            pltpu.async_copy(rows[b], acc.at[didx.at[j]], ssem[b], add=True)

        def swait(j, b):
            pltpu.make_async_copy(rows[b], acc.at[didx.at[j]], ssem[b]).wait()

        def group(g, carry):
            ja = g * 2 * _NBUF
            jb = ja + _NBUF
            for b in range(_NBUF):
                gather(ja + b, b)
            for b in range(_NBUF):
                gwait_scat(ja + b, b)
            for b in range(_NBUF):
                swait(ja + b, b)
                gather(jb + b, b)
            for b in range(_NBUF):
                gwait_scat(jb + b, b)
            for b in range(_NBUF):
                swait(jb + b, b)
            return carry

        lax.fori_loop(0, G, group, 0)
        plsc.subcore_barrier()
        pltpu.sync_copy(acc.at[pl.ds(rbase, _RPT)],
                        out.at[pl.ds(rbase, _RPT), pl.ds(c * D, D)])

    return agg


@functools.lru_cache(maxsize=None)
def _make_agg(D):
    G = _NCH // _NBUF

    @functools.partial(
        pl.kernel,
        out_type=jax.ShapeDtypeStruct((_NACC, _D), jnp.float32),
        mesh=_sc_mesh(),
        compiler_params=pltpu.CompilerParams(use_tc_tiling_on_sc=(D % 128 == 0)),
        scratch_types=(
            [pltpu.VMEM((_NCH, _C), jnp.int32)] * 2
            + [pltpu.VMEM((_C, D), jnp.float32)] * _NBUF
            + [pltpu.VMEM_SHARED((_NACC, D), jnp.float32)]
            + [pltpu.SemaphoreType.DMA] * (2 * _NBUF)
        ),
    )
    def agg(table, srcp2, dstp2, zeros, out, sidx, didx, *scr):
        rows = scr[:_NBUF]
        acc = scr[_NBUF]
        gsem = scr[_NBUF + 1:2 * _NBUF + 1]
        ssem = scr[2 * _NBUF + 1:]
        c = lax.axis_index("c")
        s = lax.axis_index("s")
        wid = c * _NS + s
        rbase = s * _RPT
        pltpu.sync_copy(zeros.at[pl.ds(rbase, _RPT)], acc.at[pl.ds(rbase, _RPT)])
        pltpu.sync_copy(srcp2.at[pl.ds(wid * _NCH, _NCH)], sidx)
        pltpu.sync_copy(dstp2.at[pl.ds(wid * _NCH, _NCH)], didx)
        plsc.subcore_barrier()

        def gather(j, b):
            pltpu.async_copy(table.at[sidx.at[j]], rows[b], gsem[b])

        def gwait_scat(j, b):
            pltpu.make_async_copy(table.at[sidx.at[j]], rows[b], gsem[b]).wait()
            pltpu.async_copy(rows[b], acc.at[didx.at[j]], ssem[b], add=True)

        def swait(j, b):
            pltpu.make_async_copy(rows[b], acc.at[didx.at[j]], ssem[b]).wait()

        def group(g, carry):
            ja = g * 2 * _NBUF
            jb = ja + _NBUF
            for b in range(_NBUF):
                gather(ja + b, b)
            for b in range(_NBUF):
                gwait_scat(ja + b, b)
            for b in range(_NBUF):
                swait(ja + b, b)
                gather(jb + b, b)
            for b in range(_NBUF):
                gwait_scat(jb + b, b)
            for b in range(_NBUF):
                swait(jb + b, b)
            return carry

        lax.fori_loop(0, _NCH // (2 * _NBUF), group, 0)
        plsc.subcore_barrier()
        pltpu.sync_copy(acc.at[pl.ds(rbase, _RPT)],
                        out.at[pl.ds(rbase, _RPT), pl.ds(c * 64, D)])

    return agg


_DEGK = 8


@functools.partial(
    pl.kernel,
    out_type=jax.ShapeDtypeStruct((_NACC, 128), jnp.float32),
    mesh=_sc_mesh(),
    compiler_params=pltpu.CompilerParams(use_tc_tiling_on_sc=False),
    scratch_types=[
        pltpu.VMEM((_NCHF, _C), jnp.int32),
        pltpu.VMEM((_C, 16), jnp.float32),
        pltpu.VMEM_SHARED((_NACC, 16), jnp.float32),
        pltpu.SemaphoreType.DMA,
        pltpu.SemaphoreType.DMA,
    ],
)
def _deg_kernel(srcp2, dstp2, ones, zeros, out, idxbuf, ones_v,
                acc, sem0, sem1):
    sems = (sem0, sem1)
    c = lax.axis_index("c")
    s = lax.axis_index("s")
    rbase = s * _RPT
    pltpu.sync_copy(ones, ones_v)
    pltpu.sync_copy(zeros.at[pl.ds(rbase, _RPT)], acc.at[pl.ds(rbase, _RPT)])

    def load_idx(idx2):
        pltpu.sync_copy(idx2.at[pl.ds(s * _NCHF, _NCHF)], idxbuf)

    pl.when(c == 0)(lambda: load_idx(srcp2))
    pl.when(c == 1)(lambda: load_idx(dstp2))
    plsc.subcore_barrier()
    NG = _NCHF // _DEGK

    def dgroup(g, carry):
        for k in range(_DEGK):
            pltpu.async_copy(ones_v, acc.at[idxbuf.at[g * _DEGK + k]],
                             sems[k % 2], add=True)
        for k in range(_DEGK):
            pltpu.make_async_copy(
                ones_v, acc.at[idxbuf.at[g * _DEGK + k]], sems[k % 2]).wait()
        return carry

    lax.fori_loop(0, NG, dgroup, 0)
    plsc.subcore_barrier()
    pltpu.sync_copy(acc.at[pl.ds(rbase, _RPT)],
                    out.at[pl.ds(rbase, _RPT), pl.ds(c * 64, 16)])


def _tc_norm_scale(xpad, deg):
    def body(x_ref, d_ref, xs_ref, ns_ref, nd_ref):
        ns = lax.rsqrt(jnp.maximum(d_ref[:, 0:1], 1.0))
        nd = lax.rsqrt(jnp.maximum(d_ref[:, 64:65], 1.0))
        xs_ref[...] = x_ref[...] * ns
        ns_ref[...] = ns
        nd_ref[...] = nd

    return pl.pallas_call(
        body,
        grid=(_GRID,),
        in_specs=[
            pl.BlockSpec((_BR, _D), lambda i: (i, 0)),
            pl.BlockSpec((_BR, 128), lambda i: (i, 0)),
        ],
        out_specs=[
            pl.BlockSpec((_BR, _D), lambda i: (i, 0)),
            pl.BlockSpec((_BR, 1), lambda i: (i, 0)),
            pl.BlockSpec((_BR, 1), lambda i: (i, 0)),
        ],
        out_shape=[
            jax.ShapeDtypeStruct((_NACC, _D), jnp.float32),
            jax.ShapeDtypeStruct((_NACC, 1), jnp.float32),
            jax.ShapeDtypeStruct((_NACC, 1), jnp.float32),
        ],
    )(xpad, deg)


def _tc_layer(p, nd, ns, W, b):
    def body(p_ref, nd_ref, ns_ref, w_ref, b_ref, o_ref):
        h = p_ref[...] * nd_ref[...]
        z = jnp.dot(h, w_ref[...], preferred_element_type=jnp.float32) + b_ref[...]
        o_ref[...] = jnp.where(z > 0.0, z, jnp.exp(z) - 1.0) * ns_ref[...]

    return pl.pallas_call(
        body,
        grid=(_GRID,),
        in_specs=[
            pl.BlockSpec((_BR, _D), lambda i: (i, 0)),
            pl.BlockSpec((_BR, 1), lambda i: (i, 0)),
            pl.BlockSpec((_BR, 1), lambda i: (i, 0)),
            pl.BlockSpec((_D, _D), lambda i: (0, 0)),
            pl.BlockSpec((1, _D), lambda i: (0, 0)),
        ],
        out_specs=pl.BlockSpec((_BR, _D), lambda i: (i, 0)),
        out_shape=jax.ShapeDtypeStruct((_NACC, _D), jnp.float32),
    )(p, nd, ns, W, b)


def _tc_layer_premat(p, nd, ns, W, b, W2p):
    def body(p_ref, nd_ref, ns_ref, w_ref, b_ref, w2_ref, o_ref):
        h = p_ref[...] * nd_ref[...]
        z = jnp.dot(h, w_ref[...], preferred_element_type=jnp.float32) + b_ref[...]
        h2 = jnp.where(z > 0.0, z, jnp.exp(z) - 1.0) * ns_ref[...]
        o_ref[...] = jnp.dot(h2, w2_ref[...], preferred_element_type=jnp.float32)

    return pl.pallas_call(
        body,
        grid=(_GRID,),
        in_specs=[
            pl.BlockSpec((_BR, _D), lambda i: (i, 0)),
            pl.BlockSpec((_BR, 1), lambda i: (i, 0)),
            pl.BlockSpec((_BR, 1), lambda i: (i, 0)),
            pl.BlockSpec((_D, _D), lambda i: (0, 0)),
            pl.BlockSpec((1, _D), lambda i: (0, 0)),
            pl.BlockSpec((_D, _DC), lambda i: (0, 0)),
        ],
        out_specs=pl.BlockSpec((_BR, _DC), lambda i: (i, 0)),
        out_shape=jax.ShapeDtypeStruct((_NACC, _DC), jnp.float32),
    )(p, nd, ns, W, b, W2p)


def _tc_final(p2, nd, b2p):
    def body(p_ref, nd_ref, b_ref, o_ref):
        p = p_ref[...]
        z = (p[:, :_DC] + p[:, 64:64 + _DC]) * nd_ref[...] + b_ref[...]
        o_ref[...] = z[:, :_NCLS]

    return pl.pallas_call(
        body,
        grid=(_GRID,),
        in_specs=[
            pl.BlockSpec((_BR, 128), lambda i: (i, 0)),
            pl.BlockSpec((_BR, 1), lambda i: (i, 0)),
            pl.BlockSpec((1, _DC), lambda i: (0, 0)),
        ],
        out_specs=pl.BlockSpec((_BR, _NCLS), lambda i: (i, 0)),
        out_shape=jax.ShapeDtypeStruct((_N, _NCLS), jnp.float32),
    )(p2, nd, b2p)


def kernel(x, edge_index, W0, b0, W1, b1, W2, b2):
    src = edge_index[0]
    dst = edge_index[1]
    padi = _N + (jnp.arange(_EPAD - _E, dtype=jnp.int32) % (_NACC - _N))
    srcp2 = jnp.concatenate([src, padi]).reshape(_NW * _NCH, _C)
    dstp2 = jnp.concatenate([dst, padi]).reshape(_NW * _NCH, _C)
    ones16 = jnp.ones((_C, 16), jnp.float32)
    zdeg = jnp.zeros((_NACC, 16), jnp.float32)
    z64 = jnp.zeros((_NACC, _D // 2), jnp.float32)
    z48 = jnp.zeros((_NACC, _DC), jnp.float32)

    srcp = jnp.concatenate([src, padi])
    s2lo = (2 * srcp).reshape(_NW * _NCH, _C)
    s2hi = (2 * srcp + 1).reshape(_NW * _NCH, _C)

    deg = _deg_kernel(srcp2, dstp2, ones16, zdeg)
    xpad = jnp.pad(x, ((0, _NACC - _N), (0, 0)))
    xs, ns, nd = _tc_norm_scale(xpad, deg)

    aggf = _agg_feature_split()
    p0 = aggf(xs.reshape(2 * _NACC, _D // 2), s2lo, s2hi, dstp2, z64)
    h1 = _tc_layer(p0, nd, ns, W0, b0.reshape(1, _D))
    p1 = aggf(h1.reshape(2 * _NACC, _D // 2), s2lo, s2hi, dstp2, z64)
    W2p = jnp.pad(W2, ((0, 0), (0, _DC - _NCLS)))
    z2 = _tc_layer_premat(p1, nd, ns, W1, b1.reshape(1, _D), W2p)
    p2 = _make_agg(_DC)(z2, srcp2, dstp2, z48)
    b2p = jnp.pad(b2, (0, _DC - _NCLS)).reshape(1, _DC)
    return _tc_final(p2, nd, b2p)
